# Initial kernel scaffold; baseline (speedup 1.0000x reference)
import jax
import jax.numpy as jnp
from jax.experimental import pallas as pl


def kernel(x, pos, edge_index, batch, W_emb, b_emb, Wq0, Wk0, Wv0, R1_0, R2_0, Rv_0, Wsh0, Wq1, Wk1, Wv1, R1_1, R2_1, Rv_1, Wsh1):
    raise NotImplementedError("write your pallas kernel here")


# speedup vs baseline: 2.2364x; 2.2364x over previous
import functools

import jax
import jax.numpy as jnp
from jax import lax
from jax.experimental import pallas as pl
from jax.experimental.pallas import tpu as pltpu
from jax.experimental.pallas import tpu_sc as plsc

N = 10000
E = 320000
G = 64
DH = 128
NB = 16
RMAX = 2.5
PI = 3.14159265358979

N_PAD = 10240
BN = 1024
C = 128
NW = 32
CHUNKS_PER_W = 79
E_PAD = C * NW * CHUNKS_PER_W
BE = 512
TD = 128
TDT = 256
TST = 384

_HI = lax.Precision.HIGHEST


def _silu(x):
  return x * (1.0 / (1.0 + jnp.exp(-x)))


def _node0_body(x_ref, pos_ref, wemb_ref, bemb_ref, wq_ref, wk_ref, wv_ref,
                tdst_ref, tsrc_ref):
  h = jnp.dot(x_ref[...], wemb_ref[...], precision=_HI,
              preferred_element_type=jnp.float32) + bemb_ref[...]
  q = jnp.dot(h, wq_ref[...], precision=_HI, preferred_element_type=jnp.float32)
  k = jnp.dot(h, wk_ref[...], precision=_HI, preferred_element_type=jnp.float32)
  v = jnp.dot(h, wv_ref[...], precision=_HI, preferred_element_type=jnp.float32)
  p = pos_ref[...]
  zq = jnp.zeros((q.shape[0], TDT - DH - 16), jnp.float32)
  zk = jnp.zeros((q.shape[0], TST - 2 * DH - 16), jnp.float32)
  tdst_ref[...] = jnp.concatenate([q, p, zq], axis=1)
  tsrc_ref[...] = jnp.concatenate([k, v, p, zk], axis=1)


def _node0(x_pad, pos16, wemb, bemb, wq, wk, wv):
  grid = N_PAD // BN
  return pl.pallas_call(
      _node0_body,
      grid=(grid,),
      in_specs=[
          pl.BlockSpec((BN, 4), lambda i: (i, 0)),
          pl.BlockSpec((BN, 16), lambda i: (i, 0)),
          pl.BlockSpec((4, 64), lambda i: (0, 0)),
          pl.BlockSpec((1, 64), lambda i: (0, 0)),
          pl.BlockSpec((64, DH), lambda i: (0, 0)),
          pl.BlockSpec((64, DH), lambda i: (0, 0)),
          pl.BlockSpec((64, DH), lambda i: (0, 0)),
      ],
      out_specs=[
          pl.BlockSpec((BN, TDT), lambda i: (i, 0)),
          pl.BlockSpec((BN, TST), lambda i: (i, 0)),
      ],
      out_shape=[
          jax.ShapeDtypeStruct((N_PAD, TDT), jnp.float32),
          jax.ShapeDtypeStruct((N_PAD, TST), jnp.float32),
      ],
  )(x_pad, pos16, wemb, bemb, wq, wk, wv)


def _node1_body(p_ref, pos16_ref, wq_ref, wk_ref, wv_ref, tdst_ref, tsrc_ref):
  p0 = p_ref[0]
  p1 = p_ref[1]
  den = p0[:, 64:65]
  h = jnp.concatenate([p0[:, :64], p1[:, :64]], axis=1) * (1.0 / (den + 1e-9))
  q = jnp.dot(h, wq_ref[...], precision=_HI, preferred_element_type=jnp.float32)
  k = jnp.dot(h, wk_ref[...], precision=_HI, preferred_element_type=jnp.float32)
  v = jnp.dot(h, wv_ref[...], precision=_HI, preferred_element_type=jnp.float32)
  pos = pos16_ref[...]
  zq = jnp.zeros((q.shape[0], TDT - DH - 16), jnp.float32)
  zk = jnp.zeros((q.shape[0], TST - 2 * DH - 16), jnp.float32)
  tdst_ref[...] = jnp.concatenate([q, pos, zq], axis=1)
  tsrc_ref[...] = jnp.concatenate([k, v, pos, zk], axis=1)


def _node1(partials, pos16, wq, wk, wv):
  grid = N_PAD // BN
  return pl.pallas_call(
      _node1_body,
      grid=(grid,),
      in_specs=[
          pl.BlockSpec((2, BN, TD), lambda i: (0, i, 0)),
          pl.BlockSpec((BN, 16), lambda i: (i, 0)),
          pl.BlockSpec((DH, DH), lambda i: (0, 0)),
          pl.BlockSpec((DH, DH), lambda i: (0, 0)),
          pl.BlockSpec((DH, DH), lambda i: (0, 0)),
      ],
      out_specs=[
          pl.BlockSpec((BN, TDT), lambda i: (i, 0)),
          pl.BlockSpec((BN, TST), lambda i: (i, 0)),
      ],
      out_shape=[
          jax.ShapeDtypeStruct((N_PAD, TDT), jnp.float32),
          jax.ShapeDtypeStruct((N_PAD, TST), jnp.float32),
      ],
  )(partials, pos16, wq, wk, wv)


def _sc_gather_body(tdst_ref, tsrc_ref, dst_ref, src_ref, edst_ref, esrc_ref,
                    idxd, idxs, bufd, bufs):
  c = lax.axis_index("c")
  s = lax.axis_index("s")
  w = s * 2 + c

  def body(i, carry):
    off = (i * NW + w) * C
    pltpu.sync_copy(dst_ref.at[pl.ds(off, C)], idxd)
    pltpu.sync_copy(src_ref.at[pl.ds(off, C)], idxs)
    pltpu.sync_copy(tdst_ref.at[idxd], bufd)
    pltpu.sync_copy(tsrc_ref.at[idxs], bufs)
    pltpu.sync_copy(bufd, edst_ref.at[pl.ds(off, C)])
    pltpu.sync_copy(bufs, esrc_ref.at[pl.ds(off, C)])
    return carry

  lax.fori_loop(0, CHUNKS_PER_W, body, 0)


def _sc_gather(tdst, tsrc, dst_pad, src_pad):
  mesh = plsc.VectorSubcoreMesh(core_axis_name="c", subcore_axis_name="s")
  f = pl.kernel(
      _sc_gather_body,
      out_type=[
          jax.ShapeDtypeStruct((E_PAD, TDT), jnp.float32),
          jax.ShapeDtypeStruct((E_PAD, TST), jnp.float32),
      ],
      mesh=mesh,
      scratch_types=[
          pltpu.VMEM((C,), jnp.int32),
          pltpu.VMEM((C,), jnp.int32),
          pltpu.VMEM((C, TDT), jnp.float32),
          pltpu.VMEM((C, TST), jnp.float32),
      ],
  )
  return f(tdst, tsrc, dst_pad, src_pad)


def _edge_body(ed_ref, es_ref, r1_ref, r2_ref, rv_ref, wsh_ref, out_ref):
  ed = ed_ref[...]
  es = es_ref[...]
  q = ed[:, :128]
  posd = ed[:, 128:131]
  k0 = es[:, :128]
  v0 = es[:, 128:256]
  poss = es[:, 256:259]

  rel = poss - posd
  r2sum = jnp.sum(rel * rel, axis=1, keepdims=True) + 1e-12
  r = jnp.sqrt(r2sum)
  dirs = rel * (1.0 / (r + 1e-9))

  centers = lax.broadcasted_iota(jnp.int32, (1, NB), 1).astype(jnp.float32) * (
      RMAX / (NB - 1))
  width = RMAX / NB
  t = (r - centers) * (1.0 / width)
  rbf = jnp.exp(-(t * t))
  env = jnp.where(r < RMAX, 0.5 * (jnp.cos(PI / RMAX * r) + 1.0), 0.0)
  rb = rbf * env

  hidden = _silu(jnp.dot(rb, r1_ref[...], precision=_HI,
                         preferred_element_type=jnp.float32))
  rk = jnp.dot(hidden, r2_ref[...], precision=_HI,
               preferred_element_type=jnp.float32)
  rvv = jnp.dot(hidden, rv_ref[...], precision=_HI,
                preferred_element_type=jnp.float32)

  x = dirs[:, 0:1]
  y = dirs[:, 1:2]
  z = dirs[:, 2:3]
  one = jnp.ones_like(x)
  sh = jnp.concatenate([
      one, x, y, z,
      1.7320508 * x * y, 1.7320508 * y * z,
      0.5 * (3.0 * z * z - 1.0),
      1.7320508 * x * z, 0.8660254 * (x * x - y * y),
      jnp.zeros((x.shape[0], 7), jnp.float32),
  ], axis=1)
  shw = jnp.dot(sh, wsh_ref[...], precision=_HI,
                preferred_element_type=jnp.float32)

  k = k0 * rk + shw
  v = v0 * rvv
  logits = jnp.sum(q * k, axis=1, keepdims=True) * (DH ** -0.5)

  i = pl.program_id(0)
  eid = lax.broadcasted_iota(jnp.int32, (BE, 1), 0) + i * BE
  ex = jnp.where(eid < E, jnp.exp(logits), 0.0)

  zpad = jnp.zeros((BE, 63), jnp.float32)
  out_ref[0] = jnp.concatenate([ex * v[:, :64], ex, zpad], axis=1)
  out_ref[1] = jnp.concatenate([ex * v[:, 64:], ex, zpad], axis=1)


def _edge(edst, esrc, r1, r2, rv, wsh16):
  grid = E_PAD // BE
  return pl.pallas_call(
      _edge_body,
      grid=(grid,),
      in_specs=[
          pl.BlockSpec((BE, TDT), lambda i: (i, 0)),
          pl.BlockSpec((BE, TST), lambda i: (i, 0)),
          pl.BlockSpec((NB, 64), lambda i: (0, 0)),
          pl.BlockSpec((64, DH), lambda i: (0, 0)),
          pl.BlockSpec((64, DH), lambda i: (0, 0)),
          pl.BlockSpec((16, DH), lambda i: (0, 0)),
      ],
      out_specs=pl.BlockSpec((2, BE, TD), lambda i: (0, i, 0)),
      out_shape=jax.ShapeDtypeStruct((2, E_PAD, TD), jnp.float32),
  )(edst, esrc, r1, r2, rv, wsh16)


def _sc_scatter_body(ev_ref, dst_ref, zrows_ref, out_ref, idxb, buf, acc):
  c = lax.axis_index("c")
  s = lax.axis_index("s")
  rpt = N_PAD // 16
  base = s * rpt
  pltpu.sync_copy(zrows_ref.at[pl.ds(0, rpt)], acc.at[pl.ds(base, rpt)])
  plsc.subcore_barrier()

  def body(i, carry):
    off = (i * 16 + s) * C
    pltpu.sync_copy(dst_ref.at[pl.ds(off, C)], idxb)
    pltpu.sync_copy(ev_ref.at[c, pl.ds(off, C)], buf)
    pltpu.sync_copy(buf, acc.at[idxb], add=True)
    return carry

  lax.fori_loop(0, CHUNKS_PER_W * 2, body, 0)
  plsc.subcore_barrier()
  pltpu.sync_copy(acc.at[pl.ds(base, rpt)], out_ref.at[c, pl.ds(base, rpt)])


def _sc_scatter(ev, dst_pad, zrows):
  mesh = plsc.VectorSubcoreMesh(core_axis_name="c", subcore_axis_name="s")
  f = pl.kernel(
      _sc_scatter_body,
      out_type=jax.ShapeDtypeStruct((2, N_PAD, TD), jnp.float32),
      mesh=mesh,
      scratch_types=[
          pltpu.VMEM((C,), jnp.int32),
          pltpu.VMEM((C, TD), jnp.float32),
          pltpu.VMEM_SHARED((N_PAD, TD), jnp.float32),
      ],
  )
  return f(ev, dst_pad, zrows)


def _pool_body(p_ref, batch_ref, out_ref, acc):
  i = pl.program_id(0)

  @pl.when(i == 0)
  def _():
    acc[...] = jnp.zeros_like(acc)

  p0 = p_ref[0]
  p1 = p_ref[1]
  den = p0[:, 64:65]
  h = jnp.concatenate([p0[:, :64], p1[:, :64]], axis=1) * (1.0 / (den + 1e-9))

  bt = batch_ref[0]
  oh = (lax.broadcasted_iota(jnp.int32, (G, BN), 0) == bt).astype(jnp.float32)
  hext = jnp.concatenate(
      [h, jnp.ones((BN, 1), jnp.float32), jnp.zeros((BN, 127), jnp.float32)],
      axis=1)
  acc[...] += jnp.dot(oh, hext, precision=_HI,
                      preferred_element_type=jnp.float32)

  @pl.when(i == (N_PAD // BN) - 1)
  def _():
    cnt = acc[:, 128:129]
    out_ref[...] = acc[:, :128] * (1.0 / jnp.maximum(cnt, 1.0))


def _pool(partials, batch3):
  grid = N_PAD // BN
  return pl.pallas_call(
      _pool_body,
      grid=(grid,),
      in_specs=[
          pl.BlockSpec((2, BN, TD), lambda i: (0, i, 0)),
          pl.BlockSpec((1, 1, BN), lambda i: (i, 0, 0)),
      ],
      out_specs=pl.BlockSpec((G, DH), lambda i: (0, 0)),
      out_shape=jax.ShapeDtypeStruct((G, DH), jnp.float32),
      scratch_shapes=[pltpu.VMEM((G, 256), jnp.float32)],
      compiler_params=pltpu.CompilerParams(
          dimension_semantics=("arbitrary",)),
  )(partials, batch3)


def kernel(x, pos, edge_index, batch, W_emb, b_emb,
           Wq0, Wk0, Wv0, R1_0, R2_0, Rv_0, Wsh0,
           Wq1, Wk1, Wv1, R1_1, R2_1, Rv_1, Wsh1):
  src = edge_index[0].astype(jnp.int32)
  dst = edge_index[1].astype(jnp.int32)
  src_pad = jnp.pad(src, (0, E_PAD - E))
  dst_pad = jnp.pad(dst, (0, E_PAD - E))

  x_pad = jnp.pad(x, ((0, N_PAD - N), (0, 0)))
  pos16 = jnp.pad(pos, ((0, N_PAD - N), (0, 13)))
  batch3 = jnp.pad(batch.astype(jnp.int32), (0, N_PAD - N),
                   constant_values=G).reshape(N_PAD // BN, 1, BN)
  bemb2 = b_emb.reshape(1, 64)
  wsh0_16 = jnp.pad(Wsh0, ((0, 7), (0, 0)))
  wsh1_16 = jnp.pad(Wsh1, ((0, 7), (0, 0)))
  zrows = jnp.zeros((N_PAD // 16, TD), jnp.float32)

  tdst, tsrc = _node0(x_pad, pos16, W_emb, bemb2, Wq0, Wk0, Wv0)
  edst, esrc = _sc_gather(tdst, tsrc, dst_pad, src_pad)
  ev = _edge(edst, esrc, R1_0, R2_0, Rv_0, wsh0_16)
  part0 = _sc_scatter(ev, dst_pad, zrows)

  tdst, tsrc = _node1(part0, pos16, Wq1, Wk1, Wv1)
  edst, esrc = _sc_gather(tdst, tsrc, dst_pad, src_pad)
  ev = _edge(edst, esrc, R1_1, R2_1, Rv_1, wsh1_16)
  part1 = _sc_scatter(ev, dst_pad, zrows)

  return _pool(part1, batch3)


# speedup vs baseline: 2.3242x; 1.0393x over previous
import functools

import jax
import jax.numpy as jnp
from jax import lax
from jax.experimental import pallas as pl
from jax.experimental.pallas import tpu as pltpu
from jax.experimental.pallas import tpu_sc as plsc

N = 10000
E = 320000
G = 64
DH = 128
NB = 16
RMAX = 2.5
PI = 3.14159265358979

N_PAD = 10240
BN = 1024
C = 128
NW = 32
CHUNKS_PER_W = 79
E_PAD = C * NW * CHUNKS_PER_W
BE = 512
TD = 128
TDT = 128
TST = 256

_HI = lax.Precision.HIGHEST


def _silu(x):
  return x * (1.0 / (1.0 + jnp.exp(-x)))


def _node0_body(x_ref, wemb_ref, bemb_ref, wq_ref, wk_ref, wv_ref,
                tdst_ref, tsrc_ref):
  h = jnp.dot(x_ref[...], wemb_ref[...], precision=_HI,
              preferred_element_type=jnp.float32) + bemb_ref[...]
  q = jnp.dot(h, wq_ref[...], precision=_HI, preferred_element_type=jnp.float32)
  k = jnp.dot(h, wk_ref[...], precision=_HI, preferred_element_type=jnp.float32)
  v = jnp.dot(h, wv_ref[...], precision=_HI, preferred_element_type=jnp.float32)
  tdst_ref[...] = q
  tsrc_ref[...] = jnp.concatenate([k, v], axis=1)


def _node0(x_pad, wemb, bemb, wq, wk, wv):
  grid = N_PAD // BN
  return pl.pallas_call(
      _node0_body,
      grid=(grid,),
      in_specs=[
          pl.BlockSpec((BN, 4), lambda i: (i, 0)),
          pl.BlockSpec((4, 64), lambda i: (0, 0)),
          pl.BlockSpec((1, 64), lambda i: (0, 0)),
          pl.BlockSpec((64, DH), lambda i: (0, 0)),
          pl.BlockSpec((64, DH), lambda i: (0, 0)),
          pl.BlockSpec((64, DH), lambda i: (0, 0)),
      ],
      out_specs=[
          pl.BlockSpec((BN, TDT), lambda i: (i, 0)),
          pl.BlockSpec((BN, TST), lambda i: (i, 0)),
      ],
      out_shape=[
          jax.ShapeDtypeStruct((N_PAD, TDT), jnp.float32),
          jax.ShapeDtypeStruct((N_PAD, TST), jnp.float32),
      ],
  )(x_pad, wemb, bemb, wq, wk, wv)


def _node1_body(p_ref, wq_ref, wk_ref, wv_ref, tdst_ref, tsrc_ref):
  p0 = p_ref[0]
  p1 = p_ref[1]
  den = p0[:, 64:65]
  h = jnp.concatenate([p0[:, :64], p1[:, :64]], axis=1) * (1.0 / (den + 1e-9))
  q = jnp.dot(h, wq_ref[...], precision=_HI, preferred_element_type=jnp.float32)
  k = jnp.dot(h, wk_ref[...], precision=_HI, preferred_element_type=jnp.float32)
  v = jnp.dot(h, wv_ref[...], precision=_HI, preferred_element_type=jnp.float32)
  tdst_ref[...] = q
  tsrc_ref[...] = jnp.concatenate([k, v], axis=1)


def _node1(partials, wq, wk, wv):
  grid = N_PAD // BN
  return pl.pallas_call(
      _node1_body,
      grid=(grid,),
      in_specs=[
          pl.BlockSpec((2, BN, TD), lambda i: (0, i, 0)),
          pl.BlockSpec((DH, DH), lambda i: (0, 0)),
          pl.BlockSpec((DH, DH), lambda i: (0, 0)),
          pl.BlockSpec((DH, DH), lambda i: (0, 0)),
      ],
      out_specs=[
          pl.BlockSpec((BN, TDT), lambda i: (i, 0)),
          pl.BlockSpec((BN, TST), lambda i: (i, 0)),
      ],
      out_shape=[
          jax.ShapeDtypeStruct((N_PAD, TDT), jnp.float32),
          jax.ShapeDtypeStruct((N_PAD, TST), jnp.float32),
      ],
  )(partials, wq, wk, wv)


def _sc_geo_body(px_ref, py_ref, pz_ref, dst_ref, src_ref, rel_ref,
                 pxv, pyv, pzv, idxd, idxs, rx, ry, rz):
  c = lax.axis_index("c")
  s = lax.axis_index("s")
  w = s * 2 + c
  pltpu.sync_copy(px_ref, pxv)
  pltpu.sync_copy(py_ref, pyv)
  pltpu.sync_copy(pz_ref, pzv)

  def body(i, carry):
    off = (i * NW + w) * C
    pltpu.sync_copy(dst_ref.at[pl.ds(off, C)], idxd)
    pltpu.sync_copy(src_ref.at[pl.ds(off, C)], idxs)
    for j in range(C // 16):
      sl = pl.ds(j * 16, 16)
      i_s = idxs[sl]
      i_d = idxd[sl]
      rx[sl] = plsc.load_gather(pxv, [i_s]) - plsc.load_gather(pxv, [i_d])
      ry[sl] = plsc.load_gather(pyv, [i_s]) - plsc.load_gather(pyv, [i_d])
      rz[sl] = plsc.load_gather(pzv, [i_s]) - plsc.load_gather(pzv, [i_d])
    pltpu.sync_copy(rx, rel_ref.at[0, pl.ds(off, C)])
    pltpu.sync_copy(ry, rel_ref.at[1, pl.ds(off, C)])
    pltpu.sync_copy(rz, rel_ref.at[2, pl.ds(off, C)])
    return carry

  lax.fori_loop(0, CHUNKS_PER_W, body, 0)


def _sc_geo(px, py, pz, dst_pad, src_pad):
  mesh = plsc.VectorSubcoreMesh(core_axis_name="c", subcore_axis_name="s")
  f = pl.kernel(
      _sc_geo_body,
      out_type=jax.ShapeDtypeStruct((8, E_PAD), jnp.float32),
      mesh=mesh,
      scratch_types=[
          pltpu.VMEM((N_PAD,), jnp.float32),
          pltpu.VMEM((N_PAD,), jnp.float32),
          pltpu.VMEM((N_PAD,), jnp.float32),
          pltpu.VMEM((C,), jnp.int32),
          pltpu.VMEM((C,), jnp.int32),
          pltpu.VMEM((C,), jnp.float32),
          pltpu.VMEM((C,), jnp.float32),
          pltpu.VMEM((C,), jnp.float32),
      ],
      compiler_params=pltpu.CompilerParams(needs_layout_passes=False),
  )
  return f(px, py, pz, dst_pad, src_pad)


def _sc_gather_body(tdst_ref, tsrc_ref, dst_ref, src_ref, edst_ref, esrc_ref,
                    idxd, idxs, bufd, bufs):
  c = lax.axis_index("c")
  s = lax.axis_index("s")
  w = s * 2 + c

  def body(i, carry):
    off = (i * NW + w) * C
    pltpu.sync_copy(dst_ref.at[pl.ds(off, C)], idxd)
    pltpu.sync_copy(src_ref.at[pl.ds(off, C)], idxs)
    pltpu.sync_copy(tdst_ref.at[idxd], bufd)
    pltpu.sync_copy(tsrc_ref.at[idxs], bufs)
    pltpu.sync_copy(bufd, edst_ref.at[pl.ds(off, C)])
    pltpu.sync_copy(bufs, esrc_ref.at[pl.ds(off, C)])
    return carry

  lax.fori_loop(0, CHUNKS_PER_W, body, 0)


def _sc_gather(tdst, tsrc, dst_pad, src_pad):
  mesh = plsc.VectorSubcoreMesh(core_axis_name="c", subcore_axis_name="s")
  f = pl.kernel(
      _sc_gather_body,
      out_type=[
          jax.ShapeDtypeStruct((E_PAD, TDT), jnp.float32),
          jax.ShapeDtypeStruct((E_PAD, TST), jnp.float32),
      ],
      mesh=mesh,
      scratch_types=[
          pltpu.VMEM((C,), jnp.int32),
          pltpu.VMEM((C,), jnp.int32),
          pltpu.VMEM((C, TDT), jnp.float32),
          pltpu.VMEM((C, TST), jnp.float32),
      ],
  )
  return f(tdst, tsrc, dst_pad, src_pad)


def _edge_body(ed_ref, es_ref, rel_ref, r1_ref, r2_ref, rv_ref, wsh_ref,
               out_ref):
  q = ed_ref[...]
  es = es_ref[...]
  k0 = es[:, :128]
  v0 = es[:, 128:256]

  rel = jnp.transpose(rel_ref[...])[:, :3]
  r2sum = jnp.sum(rel * rel, axis=1, keepdims=True) + 1e-12
  r = jnp.sqrt(r2sum)
  dirs = rel * (1.0 / (r + 1e-9))

  centers = lax.broadcasted_iota(jnp.int32, (1, NB), 1).astype(jnp.float32) * (
      RMAX / (NB - 1))
  width = RMAX / NB
  t = (r - centers) * (1.0 / width)
  rbf = jnp.exp(-(t * t))
  env = jnp.where(r < RMAX, 0.5 * (jnp.cos(PI / RMAX * r) + 1.0), 0.0)
  rb = rbf * env

  hidden = _silu(jnp.dot(rb, r1_ref[...], precision=_HI,
                         preferred_element_type=jnp.float32))
  rk = jnp.dot(hidden, r2_ref[...], precision=_HI,
               preferred_element_type=jnp.float32)
  rvv = jnp.dot(hidden, rv_ref[...], precision=_HI,
                preferred_element_type=jnp.float32)

  x = dirs[:, 0:1]
  y = dirs[:, 1:2]
  z = dirs[:, 2:3]
  one = jnp.ones_like(x)
  sh = jnp.concatenate([
      one, x, y, z,
      1.7320508 * x * y, 1.7320508 * y * z,
      0.5 * (3.0 * z * z - 1.0),
      1.7320508 * x * z, 0.8660254 * (x * x - y * y),
      jnp.zeros((x.shape[0], 7), jnp.float32),
  ], axis=1)
  shw = jnp.dot(sh, wsh_ref[...], precision=_HI,
                preferred_element_type=jnp.float32)

  k = k0 * rk + shw
  v = v0 * rvv
  logits = jnp.sum(q * k, axis=1, keepdims=True) * (DH ** -0.5)

  i = pl.program_id(0)
  eid = lax.broadcasted_iota(jnp.int32, (BE, 1), 0) + i * BE
  ex = jnp.where(eid < E, jnp.exp(logits), 0.0)

  zpad = jnp.zeros((BE, 63), jnp.float32)
  out_ref[0] = jnp.concatenate([ex * v[:, :64], ex, zpad], axis=1)
  out_ref[1] = jnp.concatenate([ex * v[:, 64:], ex, zpad], axis=1)


def _edge(edst, esrc, rel8, r1, r2, rv, wsh16):
  grid = E_PAD // BE
  return pl.pallas_call(
      _edge_body,
      grid=(grid,),
      in_specs=[
          pl.BlockSpec((BE, TDT), lambda i: (i, 0)),
          pl.BlockSpec((BE, TST), lambda i: (i, 0)),
          pl.BlockSpec((8, BE), lambda i: (0, i)),
          pl.BlockSpec((NB, 64), lambda i: (0, 0)),
          pl.BlockSpec((64, DH), lambda i: (0, 0)),
          pl.BlockSpec((64, DH), lambda i: (0, 0)),
          pl.BlockSpec((16, DH), lambda i: (0, 0)),
      ],
      out_specs=pl.BlockSpec((2, BE, TD), lambda i: (0, i, 0)),
      out_shape=jax.ShapeDtypeStruct((2, E_PAD, TD), jnp.float32),
  )(edst, esrc, rel8, r1, r2, rv, wsh16)


def _sc_scatter_body(ev_ref, dst_ref, zrows_ref, out_ref, idxb, buf, acc):
  c = lax.axis_index("c")
  s = lax.axis_index("s")
  rpt = N_PAD // 16
  base = s * rpt
  pltpu.sync_copy(zrows_ref.at[pl.ds(0, rpt)], acc.at[pl.ds(base, rpt)])
  plsc.subcore_barrier()

  def body(i, carry):
    off = (i * 16 + s) * C
    pltpu.sync_copy(dst_ref.at[pl.ds(off, C)], idxb)
    pltpu.sync_copy(ev_ref.at[c, pl.ds(off, C)], buf)
    pltpu.sync_copy(buf, acc.at[idxb], add=True)
    return carry

  lax.fori_loop(0, CHUNKS_PER_W * 2, body, 0)
  plsc.subcore_barrier()
  pltpu.sync_copy(acc.at[pl.ds(base, rpt)], out_ref.at[c, pl.ds(base, rpt)])


def _sc_scatter(ev, dst_pad, zrows):
  mesh = plsc.VectorSubcoreMesh(core_axis_name="c", subcore_axis_name="s")
  f = pl.kernel(
      _sc_scatter_body,
      out_type=jax.ShapeDtypeStruct((2, N_PAD, TD), jnp.float32),
      mesh=mesh,
      scratch_types=[
          pltpu.VMEM((C,), jnp.int32),
          pltpu.VMEM((C, TD), jnp.float32),
          pltpu.VMEM_SHARED((N_PAD, TD), jnp.float32),
      ],
  )
  return f(ev, dst_pad, zrows)


def _pool_body(p_ref, batch_ref, out_ref, acc):
  i = pl.program_id(0)

  @pl.when(i == 0)
  def _():
    acc[...] = jnp.zeros_like(acc)

  p0 = p_ref[0]
  p1 = p_ref[1]
  den = p0[:, 64:65]
  h = jnp.concatenate([p0[:, :64], p1[:, :64]], axis=1) * (1.0 / (den + 1e-9))

  bt = batch_ref[0]
  oh = (lax.broadcasted_iota(jnp.int32, (G, BN), 0) == bt).astype(jnp.float32)
  hext = jnp.concatenate(
      [h, jnp.ones((BN, 1), jnp.float32), jnp.zeros((BN, 127), jnp.float32)],
      axis=1)
  acc[...] += jnp.dot(oh, hext, precision=_HI,
                      preferred_element_type=jnp.float32)

  @pl.when(i == (N_PAD // BN) - 1)
  def _():
    cnt = acc[:, 128:129]
    out_ref[...] = acc[:, :128] * (1.0 / jnp.maximum(cnt, 1.0))


def _pool(partials, batch3):
  grid = N_PAD // BN
  return pl.pallas_call(
      _pool_body,
      grid=(grid,),
      in_specs=[
          pl.BlockSpec((2, BN, TD), lambda i: (0, i, 0)),
          pl.BlockSpec((1, 1, BN), lambda i: (i, 0, 0)),
      ],
      out_specs=pl.BlockSpec((G, DH), lambda i: (0, 0)),
      out_shape=jax.ShapeDtypeStruct((G, DH), jnp.float32),
      scratch_shapes=[pltpu.VMEM((G, 256), jnp.float32)],
      compiler_params=pltpu.CompilerParams(
          dimension_semantics=("arbitrary",)),
  )(partials, batch3)


def kernel(x, pos, edge_index, batch, W_emb, b_emb,
           Wq0, Wk0, Wv0, R1_0, R2_0, Rv_0, Wsh0,
           Wq1, Wk1, Wv1, R1_1, R2_1, Rv_1, Wsh1):
  src = edge_index[0].astype(jnp.int32)
  dst = edge_index[1].astype(jnp.int32)
  src_pad = jnp.pad(src, (0, E_PAD - E))
  dst_pad = jnp.pad(dst, (0, E_PAD - E))

  x_pad = jnp.pad(x, ((0, N_PAD - N), (0, 0)))
  pos_pad = jnp.pad(pos, ((0, N_PAD - N), (0, 0)))
  px = pos_pad[:, 0]
  py = pos_pad[:, 1]
  pz = pos_pad[:, 2]
  batch3 = jnp.pad(batch.astype(jnp.int32), (0, N_PAD - N),
                   constant_values=G).reshape(N_PAD // BN, 1, BN)
  bemb2 = b_emb.reshape(1, 64)
  wsh0_16 = jnp.pad(Wsh0, ((0, 7), (0, 0)))
  wsh1_16 = jnp.pad(Wsh1, ((0, 7), (0, 0)))
  zrows = jnp.zeros((N_PAD // 16, TD), jnp.float32)

  rel8 = _sc_geo(px, py, pz, dst_pad, src_pad)

  tdst, tsrc = _node0(x_pad, W_emb, bemb2, Wq0, Wk0, Wv0)
  edst, esrc = _sc_gather(tdst, tsrc, dst_pad, src_pad)
  ev = _edge(edst, esrc, rel8, R1_0, R2_0, Rv_0, wsh0_16)
  part0 = _sc_scatter(ev, dst_pad, zrows)

  tdst, tsrc = _node1(part0, Wq1, Wk1, Wv1)
  edst, esrc = _sc_gather(tdst, tsrc, dst_pad, src_pad)
  ev = _edge(edst, esrc, rel8, R1_1, R2_1, Rv_1, wsh1_16)
  part1 = _sc_scatter(ev, dst_pad, zrows)

  return _pool(part1, batch3)


# speedup vs baseline: 3.0077x; 1.2941x over previous
import functools

import jax
import jax.numpy as jnp
from jax import lax
from jax.experimental import pallas as pl
from jax.experimental.pallas import tpu as pltpu
from jax.experimental.pallas import tpu_sc as plsc

N = 10000
E = 320000
G = 64
DH = 128
NB = 16
RMAX = 2.5
PI = 3.14159265358979

N_PAD = 10240
BN = 1024
C = 128
NW = 32
CHUNKS_PER_W = 79
E_PAD = C * NW * CHUNKS_PER_W
BE = 512
TD = 128
TDT = 128
TST = 256

_HI = lax.Precision.HIGHEST


def _silu(x):
  return x * (1.0 / (1.0 + jnp.exp(-x)))


def _node0_body(x_ref, wemb_ref, bemb_ref, wq_ref, wk_ref, wv_ref,
                tdst_ref, tsrc_ref):
  h = jnp.dot(x_ref[...], wemb_ref[...], precision=_HI,
              preferred_element_type=jnp.float32) + bemb_ref[...]
  q = jnp.dot(h, wq_ref[...], precision=_HI, preferred_element_type=jnp.float32)
  k = jnp.dot(h, wk_ref[...], precision=_HI, preferred_element_type=jnp.float32)
  v = jnp.dot(h, wv_ref[...], precision=_HI, preferred_element_type=jnp.float32)
  tdst_ref[...] = q
  tsrc_ref[...] = jnp.concatenate([k, v], axis=1)


def _node0(x_pad, wemb, bemb, wq, wk, wv):
  grid = N_PAD // BN
  return pl.pallas_call(
      _node0_body,
      grid=(grid,),
      in_specs=[
          pl.BlockSpec((BN, 4), lambda i: (i, 0)),
          pl.BlockSpec((4, 64), lambda i: (0, 0)),
          pl.BlockSpec((1, 64), lambda i: (0, 0)),
          pl.BlockSpec((64, DH), lambda i: (0, 0)),
          pl.BlockSpec((64, DH), lambda i: (0, 0)),
          pl.BlockSpec((64, DH), lambda i: (0, 0)),
      ],
      out_specs=[
          pl.BlockSpec((BN, TDT), lambda i: (i, 0)),
          pl.BlockSpec((BN, TST), lambda i: (i, 0)),
      ],
      out_shape=[
          jax.ShapeDtypeStruct((N_PAD, TDT), jnp.float32),
          jax.ShapeDtypeStruct((N_PAD, TST), jnp.float32),
      ],
  )(x_pad, wemb, bemb, wq, wk, wv)


def _node1_body(p_ref, wq_ref, wk_ref, wv_ref, tdst_ref, tsrc_ref):
  p0 = p_ref[0]
  p1 = p_ref[1]
  den = p0[:, 64:65]
  h = jnp.concatenate([p0[:, :64], p1[:, :64]], axis=1) * (1.0 / (den + 1e-9))
  q = jnp.dot(h, wq_ref[...], precision=_HI, preferred_element_type=jnp.float32)
  k = jnp.dot(h, wk_ref[...], precision=_HI, preferred_element_type=jnp.float32)
  v = jnp.dot(h, wv_ref[...], precision=_HI, preferred_element_type=jnp.float32)
  tdst_ref[...] = q
  tsrc_ref[...] = jnp.concatenate([k, v], axis=1)


def _node1(partials, wq, wk, wv):
  grid = N_PAD // BN
  return pl.pallas_call(
      _node1_body,
      grid=(grid,),
      in_specs=[
          pl.BlockSpec((2, BN, TD), lambda i: (0, i, 0)),
          pl.BlockSpec((DH, DH), lambda i: (0, 0)),
          pl.BlockSpec((DH, DH), lambda i: (0, 0)),
          pl.BlockSpec((DH, DH), lambda i: (0, 0)),
      ],
      out_specs=[
          pl.BlockSpec((BN, TDT), lambda i: (i, 0)),
          pl.BlockSpec((BN, TST), lambda i: (i, 0)),
      ],
      out_shape=[
          jax.ShapeDtypeStruct((N_PAD, TDT), jnp.float32),
          jax.ShapeDtypeStruct((N_PAD, TST), jnp.float32),
      ],
  )(partials, wq, wk, wv)


def _sc_geo_body(px_ref, py_ref, pz_ref, dst_ref, src_ref, rel_ref,
                 pxv, pyv, pzv, idxd, idxs, rx, ry, rz):
  c = lax.axis_index("c")
  s = lax.axis_index("s")
  w = s * 2 + c
  pltpu.sync_copy(px_ref, pxv)
  pltpu.sync_copy(py_ref, pyv)
  pltpu.sync_copy(pz_ref, pzv)

  def body(i, carry):
    off = (i * NW + w) * C
    pltpu.sync_copy(dst_ref.at[pl.ds(off, C)], idxd)
    pltpu.sync_copy(src_ref.at[pl.ds(off, C)], idxs)
    for j in range(C // 16):
      sl = pl.ds(j * 16, 16)
      i_s = idxs[sl]
      i_d = idxd[sl]
      rx[sl] = plsc.load_gather(pxv, [i_s]) - plsc.load_gather(pxv, [i_d])
      ry[sl] = plsc.load_gather(pyv, [i_s]) - plsc.load_gather(pyv, [i_d])
      rz[sl] = plsc.load_gather(pzv, [i_s]) - plsc.load_gather(pzv, [i_d])
    pltpu.sync_copy(rx, rel_ref.at[0, pl.ds(off, C)])
    pltpu.sync_copy(ry, rel_ref.at[1, pl.ds(off, C)])
    pltpu.sync_copy(rz, rel_ref.at[2, pl.ds(off, C)])
    return carry

  lax.fori_loop(0, CHUNKS_PER_W, body, 0)


def _sc_geo(px, py, pz, dst_pad, src_pad):
  mesh = plsc.VectorSubcoreMesh(core_axis_name="c", subcore_axis_name="s")
  f = pl.kernel(
      _sc_geo_body,
      out_type=jax.ShapeDtypeStruct((8, E_PAD), jnp.float32),
      mesh=mesh,
      scratch_types=[
          pltpu.VMEM((N_PAD,), jnp.float32),
          pltpu.VMEM((N_PAD,), jnp.float32),
          pltpu.VMEM((N_PAD,), jnp.float32),
          pltpu.VMEM((C,), jnp.int32),
          pltpu.VMEM((C,), jnp.int32),
          pltpu.VMEM((C,), jnp.float32),
          pltpu.VMEM((C,), jnp.float32),
          pltpu.VMEM((C,), jnp.float32),
      ],
      compiler_params=pltpu.CompilerParams(needs_layout_passes=False),
  )
  return f(px, py, pz, dst_pad, src_pad)


def _sc_gather_body(tdst_ref, tsrc_ref, dst_ref, src_ref, edst_ref, esrc_ref,
                    idxd, idxs, bufd, bufs):
  c = lax.axis_index("c")
  s = lax.axis_index("s")
  w = s * 2 + c

  def body(i, carry):
    off = (i * NW + w) * C
    pltpu.sync_copy(dst_ref.at[pl.ds(off, C)], idxd)
    pltpu.sync_copy(src_ref.at[pl.ds(off, C)], idxs)
    pltpu.sync_copy(tdst_ref.at[idxd], bufd)
    pltpu.sync_copy(tsrc_ref.at[idxs], bufs)
    pltpu.sync_copy(bufd, edst_ref.at[pl.ds(off, C)])
    pltpu.sync_copy(bufs, esrc_ref.at[pl.ds(off, C)])
    return carry

  lax.fori_loop(0, CHUNKS_PER_W, body, 0)


def _sc_gather(tdst, tsrc, dst_pad, src_pad):
  mesh = plsc.VectorSubcoreMesh(core_axis_name="c", subcore_axis_name="s")
  f = pl.kernel(
      _sc_gather_body,
      out_type=[
          jax.ShapeDtypeStruct((E_PAD, TDT), jnp.float32),
          jax.ShapeDtypeStruct((E_PAD, TST), jnp.float32),
      ],
      mesh=mesh,
      scratch_types=[
          pltpu.VMEM((C,), jnp.int32),
          pltpu.VMEM((C,), jnp.int32),
          pltpu.VMEM((C, TDT), jnp.float32),
          pltpu.VMEM((C, TST), jnp.float32),
      ],
  )
  return f(tdst, tsrc, dst_pad, src_pad)


def _edge_body(ed_ref, es_ref, rel_ref, r1_ref, r2_ref, rv_ref, wsh_ref,
               out_ref):
  i = pl.program_id(0)

  @pl.when(i >= E // BE)
  def _():
    out_ref[...] = jnp.zeros_like(out_ref)

  @pl.when(i < E // BE)
  def _():
    q = ed_ref[...]
    es = es_ref[...]
    k0 = es[:, :128]
    v0 = es[:, 128:256]

    relT = rel_ref[...]
    rx = relT[0:1, :]
    ry = relT[1:2, :]
    rz = relT[2:3, :]
    r2T = rx * rx + ry * ry + rz * rz + 1e-12
    rT = jnp.sqrt(r2T)
    inv_r = 1.0 / (rT + 1e-9)
    dx = rx * inv_r
    dy = ry * inv_r
    dz = rz * inv_r

    centers = lax.broadcasted_iota(jnp.int32, (NB, BE), 0).astype(
        jnp.float32) * (RMAX / (NB - 1))
    width = RMAX / NB
    tT = (jnp.broadcast_to(rT, (NB, BE)) - centers) * (1.0 / width)
    rbfT = jnp.exp(-(tT * tT))
    envT = jnp.where(rT < RMAX, 0.5 * (jnp.cos(PI / RMAX * rT) + 1.0), 0.0)
    rbT = rbfT * envT
    rb = jnp.transpose(rbT)

    hidden = _silu(jnp.dot(rb, r1_ref[...], precision=_HI,
                           preferred_element_type=jnp.float32))
    rk = jnp.dot(hidden, r2_ref[...], precision=_HI,
                 preferred_element_type=jnp.float32)
    rvv = jnp.dot(hidden, rv_ref[...], precision=_HI,
                  preferred_element_type=jnp.float32)

    one = jnp.ones_like(dx)
    shT = jnp.concatenate([
        one, dx, dy, dz,
        1.7320508 * dx * dy, 1.7320508 * dy * dz,
        0.5 * (3.0 * dz * dz - 1.0),
        1.7320508 * dx * dz, 0.8660254 * (dx * dx - dy * dy),
        jnp.zeros((7, BE), jnp.float32),
    ], axis=0)
    sh = jnp.transpose(shT)
    shw = jnp.dot(sh, wsh_ref[...], precision=_HI,
                  preferred_element_type=jnp.float32)

    k = k0 * rk + shw
    v = v0 * rvv
    logits = jnp.sum(q * k, axis=1, keepdims=True) * (DH ** -0.5)
    ex = jnp.exp(logits)

    zpad = jnp.zeros((BE, 63), jnp.float32)
    out_ref[0] = jnp.concatenate([ex * v[:, :64], ex, zpad], axis=1)
    out_ref[1] = jnp.concatenate([ex * v[:, 64:], ex, zpad], axis=1)


def _edge(edst, esrc, rel8, r1, r2, rv, wsh16):
  grid = E_PAD // BE
  return pl.pallas_call(
      _edge_body,
      grid=(grid,),
      in_specs=[
          pl.BlockSpec((BE, TDT), lambda i: (i, 0)),
          pl.BlockSpec((BE, TST), lambda i: (i, 0)),
          pl.BlockSpec((8, BE), lambda i: (0, i)),
          pl.BlockSpec((NB, 64), lambda i: (0, 0)),
          pl.BlockSpec((64, DH), lambda i: (0, 0)),
          pl.BlockSpec((64, DH), lambda i: (0, 0)),
          pl.BlockSpec((16, DH), lambda i: (0, 0)),
      ],
      out_specs=pl.BlockSpec((2, BE, TD), lambda i: (0, i, 0)),
      out_shape=jax.ShapeDtypeStruct((2, E_PAD, TD), jnp.float32),
  )(edst, esrc, rel8, r1, r2, rv, wsh16)


def _sc_scatter_body(ev_ref, dst_ref, zrows_ref, out_ref, idxb, buf, acc):
  c = lax.axis_index("c")
  s = lax.axis_index("s")
  rpt = N_PAD // 16
  base = s * rpt
  pltpu.sync_copy(zrows_ref.at[pl.ds(0, rpt)], acc.at[pl.ds(base, rpt)])
  plsc.subcore_barrier()

  def body(i, carry):
    off = (i * 16 + s) * C
    pltpu.sync_copy(dst_ref.at[pl.ds(off, C)], idxb)
    pltpu.sync_copy(ev_ref.at[c, pl.ds(off, C)], buf)
    pltpu.sync_copy(buf, acc.at[idxb], add=True)
    return carry

  lax.fori_loop(0, CHUNKS_PER_W * 2, body, 0)
  plsc.subcore_barrier()
  pltpu.sync_copy(acc.at[pl.ds(base, rpt)], out_ref.at[c, pl.ds(base, rpt)])


def _sc_scatter(ev, dst_pad, zrows):
  mesh = plsc.VectorSubcoreMesh(core_axis_name="c", subcore_axis_name="s")
  f = pl.kernel(
      _sc_scatter_body,
      out_type=jax.ShapeDtypeStruct((2, N_PAD, TD), jnp.float32),
      mesh=mesh,
      scratch_types=[
          pltpu.VMEM((C,), jnp.int32),
          pltpu.VMEM((C, TD), jnp.float32),
          pltpu.VMEM_SHARED((N_PAD, TD), jnp.float32),
      ],
  )
  return f(ev, dst_pad, zrows)


def _pool_body(p_ref, batch_ref, out_ref, acc):
  i = pl.program_id(0)

  @pl.when(i == 0)
  def _():
    acc[...] = jnp.zeros_like(acc)

  p0 = p_ref[0]
  p1 = p_ref[1]
  den = p0[:, 64:65]
  h = jnp.concatenate([p0[:, :64], p1[:, :64]], axis=1) * (1.0 / (den + 1e-9))

  bt = batch_ref[0]
  oh = (lax.broadcasted_iota(jnp.int32, (G, BN), 0) == bt).astype(jnp.float32)
  hext = jnp.concatenate(
      [h, jnp.ones((BN, 1), jnp.float32), jnp.zeros((BN, 127), jnp.float32)],
      axis=1)
  acc[...] += jnp.dot(oh, hext, precision=_HI,
                      preferred_element_type=jnp.float32)

  @pl.when(i == (N_PAD // BN) - 1)
  def _():
    cnt = acc[:, 128:129]
    out_ref[...] = acc[:, :128] * (1.0 / jnp.maximum(cnt, 1.0))


def _pool(partials, batch3):
  grid = N_PAD // BN
  return pl.pallas_call(
      _pool_body,
      grid=(grid,),
      in_specs=[
          pl.BlockSpec((2, BN, TD), lambda i: (0, i, 0)),
          pl.BlockSpec((1, 1, BN), lambda i: (i, 0, 0)),
      ],
      out_specs=pl.BlockSpec((G, DH), lambda i: (0, 0)),
      out_shape=jax.ShapeDtypeStruct((G, DH), jnp.float32),
      scratch_shapes=[pltpu.VMEM((G, 256), jnp.float32)],
      compiler_params=pltpu.CompilerParams(
          dimension_semantics=("arbitrary",)),
  )(partials, batch3)


def kernel(x, pos, edge_index, batch, W_emb, b_emb,
           Wq0, Wk0, Wv0, R1_0, R2_0, Rv_0, Wsh0,
           Wq1, Wk1, Wv1, R1_1, R2_1, Rv_1, Wsh1):
  src = edge_index[0].astype(jnp.int32)
  dst = edge_index[1].astype(jnp.int32)
  src_pad = jnp.pad(src, (0, E_PAD - E))
  dst_pad = jnp.pad(dst, (0, E_PAD - E))

  x_pad = jnp.pad(x, ((0, N_PAD - N), (0, 0)))
  pos_pad = jnp.pad(pos, ((0, N_PAD - N), (0, 0)))
  px = pos_pad[:, 0]
  py = pos_pad[:, 1]
  pz = pos_pad[:, 2]
  batch3 = jnp.pad(batch.astype(jnp.int32), (0, N_PAD - N),
                   constant_values=G).reshape(N_PAD // BN, 1, BN)
  bemb2 = b_emb.reshape(1, 64)
  wsh0_16 = jnp.pad(Wsh0, ((0, 7), (0, 0)))
  wsh1_16 = jnp.pad(Wsh1, ((0, 7), (0, 0)))
  zrows = jnp.zeros((N_PAD // 16, TD), jnp.float32)

  rel8 = _sc_geo(px, py, pz, dst_pad, src_pad)

  tdst, tsrc = _node0(x_pad, W_emb, bemb2, Wq0, Wk0, Wv0)
  edst, esrc = _sc_gather(tdst, tsrc, dst_pad, src_pad)
  ev = _edge(edst, esrc, rel8, R1_0, R2_0, Rv_0, wsh0_16)
  part0 = _sc_scatter(ev, dst_pad, zrows)

  tdst, tsrc = _node1(part0, Wq1, Wk1, Wv1)
  edst, esrc = _sc_gather(tdst, tsrc, dst_pad, src_pad)
  ev = _edge(edst, esrc, rel8, R1_1, R2_1, Rv_1, wsh1_16)
  part1 = _sc_scatter(ev, dst_pad, zrows)

  return _pool(part1, batch3)


# speedup vs baseline: 3.7181x; 1.2362x over previous
import functools

import jax
import jax.numpy as jnp
from jax import lax
from jax.experimental import pallas as pl
from jax.experimental.pallas import tpu as pltpu
from jax.experimental.pallas import tpu_sc as plsc

N = 10000
E = 320000
G = 64
DH = 128
NB = 16
RMAX = 2.5
PI = 3.14159265358979

N_PAD = 10240
BN = 1024
C = 128
NW = 32
CHUNKS_PER_W = 79
E_PAD = C * NW * CHUNKS_PER_W
BE = 512
TD = 128
TDT = 128
TST = 256

_HI = lax.Precision.DEFAULT


def _silu(x):
  return x * (1.0 / (1.0 + jnp.exp(-x)))


def _node0_body(x_ref, wemb_ref, bemb_ref, wq_ref, wk_ref, wv_ref,
                tdst_ref, tsrc_ref):
  h = jnp.dot(x_ref[...], wemb_ref[...], precision=_HI,
              preferred_element_type=jnp.float32) + bemb_ref[...]
  q = jnp.dot(h, wq_ref[...], precision=_HI, preferred_element_type=jnp.float32)
  k = jnp.dot(h, wk_ref[...], precision=_HI, preferred_element_type=jnp.float32)
  v = jnp.dot(h, wv_ref[...], precision=_HI, preferred_element_type=jnp.float32)
  tdst_ref[...] = q
  tsrc_ref[...] = jnp.concatenate([k, v], axis=1)


def _node0(x_pad, wemb, bemb, wq, wk, wv):
  grid = N_PAD // BN
  return pl.pallas_call(
      _node0_body,
      grid=(grid,),
      in_specs=[
          pl.BlockSpec((BN, 4), lambda i: (i, 0)),
          pl.BlockSpec((4, 64), lambda i: (0, 0)),
          pl.BlockSpec((1, 64), lambda i: (0, 0)),
          pl.BlockSpec((64, DH), lambda i: (0, 0)),
          pl.BlockSpec((64, DH), lambda i: (0, 0)),
          pl.BlockSpec((64, DH), lambda i: (0, 0)),
      ],
      out_specs=[
          pl.BlockSpec((BN, TDT), lambda i: (i, 0)),
          pl.BlockSpec((BN, TST), lambda i: (i, 0)),
      ],
      out_shape=[
          jax.ShapeDtypeStruct((N_PAD, TDT), jnp.float32),
          jax.ShapeDtypeStruct((N_PAD, TST), jnp.float32),
      ],
  )(x_pad, wemb, bemb, wq, wk, wv)


def _node1_body(p_ref, wq_ref, wk_ref, wv_ref, tdst_ref, tsrc_ref):
  p0 = p_ref[0]
  p1 = p_ref[1]
  den = p0[:, 64:65]
  h = jnp.concatenate([p0[:, :64], p1[:, :64]], axis=1) * (1.0 / (den + 1e-9))
  q = jnp.dot(h, wq_ref[...], precision=_HI, preferred_element_type=jnp.float32)
  k = jnp.dot(h, wk_ref[...], precision=_HI, preferred_element_type=jnp.float32)
  v = jnp.dot(h, wv_ref[...], precision=_HI, preferred_element_type=jnp.float32)
  tdst_ref[...] = q
  tsrc_ref[...] = jnp.concatenate([k, v], axis=1)


def _node1(partials, wq, wk, wv):
  grid = N_PAD // BN
  return pl.pallas_call(
      _node1_body,
      grid=(grid,),
      in_specs=[
          pl.BlockSpec((2, BN, TD), lambda i: (0, i, 0)),
          pl.BlockSpec((DH, DH), lambda i: (0, 0)),
          pl.BlockSpec((DH, DH), lambda i: (0, 0)),
          pl.BlockSpec((DH, DH), lambda i: (0, 0)),
      ],
      out_specs=[
          pl.BlockSpec((BN, TDT), lambda i: (i, 0)),
          pl.BlockSpec((BN, TST), lambda i: (i, 0)),
      ],
      out_shape=[
          jax.ShapeDtypeStruct((N_PAD, TDT), jnp.float32),
          jax.ShapeDtypeStruct((N_PAD, TST), jnp.float32),
      ],
  )(partials, wq, wk, wv)


def _sc_geo_body(px_ref, py_ref, pz_ref, dst_ref, src_ref, rel_ref,
                 pxv, pyv, pzv, idxd, idxs, rx, ry, rz):
  c = lax.axis_index("c")
  s = lax.axis_index("s")
  w = s * 2 + c
  pltpu.sync_copy(px_ref, pxv)
  pltpu.sync_copy(py_ref, pyv)
  pltpu.sync_copy(pz_ref, pzv)

  def body(i, carry):
    off = (i * NW + w) * C
    pltpu.sync_copy(dst_ref.at[pl.ds(off, C)], idxd)
    pltpu.sync_copy(src_ref.at[pl.ds(off, C)], idxs)
    for j in range(C // 16):
      sl = pl.ds(j * 16, 16)
      i_s = idxs[sl]
      i_d = idxd[sl]
      rx[sl] = plsc.load_gather(pxv, [i_s]) - plsc.load_gather(pxv, [i_d])
      ry[sl] = plsc.load_gather(pyv, [i_s]) - plsc.load_gather(pyv, [i_d])
      rz[sl] = plsc.load_gather(pzv, [i_s]) - plsc.load_gather(pzv, [i_d])
    pltpu.sync_copy(rx, rel_ref.at[0, pl.ds(off, C)])
    pltpu.sync_copy(ry, rel_ref.at[1, pl.ds(off, C)])
    pltpu.sync_copy(rz, rel_ref.at[2, pl.ds(off, C)])
    return carry

  lax.fori_loop(0, CHUNKS_PER_W, body, 0)


def _sc_geo(px, py, pz, dst_pad, src_pad):
  mesh = plsc.VectorSubcoreMesh(core_axis_name="c", subcore_axis_name="s")
  f = pl.kernel(
      _sc_geo_body,
      out_type=jax.ShapeDtypeStruct((8, E_PAD), jnp.float32),
      mesh=mesh,
      scratch_types=[
          pltpu.VMEM((N_PAD,), jnp.float32),
          pltpu.VMEM((N_PAD,), jnp.float32),
          pltpu.VMEM((N_PAD,), jnp.float32),
          pltpu.VMEM((C,), jnp.int32),
          pltpu.VMEM((C,), jnp.int32),
          pltpu.VMEM((C,), jnp.float32),
          pltpu.VMEM((C,), jnp.float32),
          pltpu.VMEM((C,), jnp.float32),
      ],
      compiler_params=pltpu.CompilerParams(needs_layout_passes=False),
  )
  return f(px, py, pz, dst_pad, src_pad)


def _sc_gather_body(tdst_ref, tsrc_ref, dst_ref, src_ref, edst_ref, esrc_ref,
                    idxd, idxs, bufd, bufs):
  c = lax.axis_index("c")
  s = lax.axis_index("s")
  w = s * 2 + c

  def body(i, carry):
    off = (i * NW + w) * C
    pltpu.sync_copy(dst_ref.at[pl.ds(off, C)], idxd)
    pltpu.sync_copy(src_ref.at[pl.ds(off, C)], idxs)
    pltpu.sync_copy(tdst_ref.at[idxd], bufd)
    pltpu.sync_copy(tsrc_ref.at[idxs], bufs)
    pltpu.sync_copy(bufd, edst_ref.at[pl.ds(off, C)])
    pltpu.sync_copy(bufs, esrc_ref.at[pl.ds(off, C)])
    return carry

  lax.fori_loop(0, CHUNKS_PER_W, body, 0)


def _sc_gather(tdst, tsrc, dst_pad, src_pad):
  mesh = plsc.VectorSubcoreMesh(core_axis_name="c", subcore_axis_name="s")
  f = pl.kernel(
      _sc_gather_body,
      out_type=[
          jax.ShapeDtypeStruct((E_PAD, TDT), jnp.float32),
          jax.ShapeDtypeStruct((E_PAD, TST), jnp.float32),
      ],
      mesh=mesh,
      scratch_types=[
          pltpu.VMEM((C,), jnp.int32),
          pltpu.VMEM((C,), jnp.int32),
          pltpu.VMEM((C, TDT), jnp.float32),
          pltpu.VMEM((C, TST), jnp.float32),
      ],
  )
  return f(tdst, tsrc, dst_pad, src_pad)


def _edge_body(ed_ref, es_ref, rel_ref, r1_ref, r2_ref, rv_ref, wsh_ref,
               out_ref):
  i = pl.program_id(0)

  @pl.when(i >= E // BE)
  def _():
    out_ref[...] = jnp.zeros_like(out_ref)

  @pl.when(i < E // BE)
  def _():
    q = ed_ref[...]
    es = es_ref[...]
    k0 = es[:, :128]
    v0 = es[:, 128:256]

    relT = rel_ref[...]
    rx = relT[0:1, :]
    ry = relT[1:2, :]
    rz = relT[2:3, :]
    r2T = rx * rx + ry * ry + rz * rz + 1e-12
    rT = jnp.sqrt(r2T)
    inv_r = 1.0 / (rT + 1e-9)
    dx = rx * inv_r
    dy = ry * inv_r
    dz = rz * inv_r

    centers = lax.broadcasted_iota(jnp.int32, (NB, BE), 0).astype(
        jnp.float32) * (RMAX / (NB - 1))
    width = RMAX / NB
    tT = (jnp.broadcast_to(rT, (NB, BE)) - centers) * (1.0 / width)
    rbfT = jnp.exp(-(tT * tT))
    envT = jnp.where(rT < RMAX, 0.5 * (jnp.cos(PI / RMAX * rT) + 1.0), 0.0)
    rbT = rbfT * envT
    rb = jnp.transpose(rbT)

    hidden = _silu(jnp.dot(rb, r1_ref[...], precision=_HI,
                           preferred_element_type=jnp.float32))
    rk = jnp.dot(hidden, r2_ref[...], precision=_HI,
                 preferred_element_type=jnp.float32)
    rvv = jnp.dot(hidden, rv_ref[...], precision=_HI,
                  preferred_element_type=jnp.float32)

    one = jnp.ones_like(dx)
    shT = jnp.concatenate([
        one, dx, dy, dz,
        1.7320508 * dx * dy, 1.7320508 * dy * dz,
        0.5 * (3.0 * dz * dz - 1.0),
        1.7320508 * dx * dz, 0.8660254 * (dx * dx - dy * dy),
        jnp.zeros((7, BE), jnp.float32),
    ], axis=0)
    sh = jnp.transpose(shT)
    shw = jnp.dot(sh, wsh_ref[...], precision=_HI,
                  preferred_element_type=jnp.float32)

    k = k0 * rk + shw
    v = v0 * rvv
    logits = jnp.sum(q * k, axis=1, keepdims=True) * (DH ** -0.5)
    ex = jnp.exp(logits)

    zpad = jnp.zeros((BE, 63), jnp.float32)
    out_ref[0] = jnp.concatenate([ex * v[:, :64], ex, zpad], axis=1)
    out_ref[1] = jnp.concatenate([ex * v[:, 64:], ex, zpad], axis=1)


def _edge(edst, esrc, rel8, r1, r2, rv, wsh16):
  grid = E_PAD // BE
  return pl.pallas_call(
      _edge_body,
      grid=(grid,),
      in_specs=[
          pl.BlockSpec((BE, TDT), lambda i: (i, 0)),
          pl.BlockSpec((BE, TST), lambda i: (i, 0)),
          pl.BlockSpec((8, BE), lambda i: (0, i)),
          pl.BlockSpec((NB, 64), lambda i: (0, 0)),
          pl.BlockSpec((64, DH), lambda i: (0, 0)),
          pl.BlockSpec((64, DH), lambda i: (0, 0)),
          pl.BlockSpec((16, DH), lambda i: (0, 0)),
      ],
      out_specs=pl.BlockSpec((2, BE, TD), lambda i: (0, i, 0)),
      out_shape=jax.ShapeDtypeStruct((2, E_PAD, TD), jnp.float32),
  )(edst, esrc, rel8, r1, r2, rv, wsh16)


def _sc_scatter_body(ev_ref, dst_ref, zrows_ref, out_ref, idxb, buf, acc):
  c = lax.axis_index("c")
  s = lax.axis_index("s")
  rpt = N_PAD // 16
  base = s * rpt
  pltpu.sync_copy(zrows_ref.at[pl.ds(0, rpt)], acc.at[pl.ds(base, rpt)])
  plsc.subcore_barrier()

  def body(i, carry):
    off = (i * 16 + s) * C
    pltpu.sync_copy(dst_ref.at[pl.ds(off, C)], idxb)
    pltpu.sync_copy(ev_ref.at[c, pl.ds(off, C)], buf)
    pltpu.sync_copy(buf, acc.at[idxb], add=True)
    return carry

  lax.fori_loop(0, CHUNKS_PER_W * 2, body, 0)
  plsc.subcore_barrier()
  pltpu.sync_copy(acc.at[pl.ds(base, rpt)], out_ref.at[c, pl.ds(base, rpt)])


def _sc_scatter(ev, dst_pad, zrows):
  mesh = plsc.VectorSubcoreMesh(core_axis_name="c", subcore_axis_name="s")
  f = pl.kernel(
      _sc_scatter_body,
      out_type=jax.ShapeDtypeStruct((2, N_PAD, TD), jnp.float32),
      mesh=mesh,
      scratch_types=[
          pltpu.VMEM((C,), jnp.int32),
          pltpu.VMEM((C, TD), jnp.float32),
          pltpu.VMEM_SHARED((N_PAD, TD), jnp.float32),
      ],
  )
  return f(ev, dst_pad, zrows)


def _pool_body(p_ref, batch_ref, out_ref, acc):
  i = pl.program_id(0)

  @pl.when(i == 0)
  def _():
    acc[...] = jnp.zeros_like(acc)

  p0 = p_ref[0]
  p1 = p_ref[1]
  den = p0[:, 64:65]
  h = jnp.concatenate([p0[:, :64], p1[:, :64]], axis=1) * (1.0 / (den + 1e-9))

  bt = batch_ref[0]
  oh = (lax.broadcasted_iota(jnp.int32, (G, BN), 0) == bt).astype(jnp.float32)
  hext = jnp.concatenate(
      [h, jnp.ones((BN, 1), jnp.float32), jnp.zeros((BN, 127), jnp.float32)],
      axis=1)
  acc[...] += jnp.dot(oh, hext, precision=_HI,
                      preferred_element_type=jnp.float32)

  @pl.when(i == (N_PAD // BN) - 1)
  def _():
    cnt = acc[:, 128:129]
    out_ref[...] = acc[:, :128] * (1.0 / jnp.maximum(cnt, 1.0))


def _pool(partials, batch3):
  grid = N_PAD // BN
  return pl.pallas_call(
      _pool_body,
      grid=(grid,),
      in_specs=[
          pl.BlockSpec((2, BN, TD), lambda i: (0, i, 0)),
          pl.BlockSpec((1, 1, BN), lambda i: (i, 0, 0)),
      ],
      out_specs=pl.BlockSpec((G, DH), lambda i: (0, 0)),
      out_shape=jax.ShapeDtypeStruct((G, DH), jnp.float32),
      scratch_shapes=[pltpu.VMEM((G, 256), jnp.float32)],
      compiler_params=pltpu.CompilerParams(
          dimension_semantics=("arbitrary",)),
  )(partials, batch3)


def kernel(x, pos, edge_index, batch, W_emb, b_emb,
           Wq0, Wk0, Wv0, R1_0, R2_0, Rv_0, Wsh0,
           Wq1, Wk1, Wv1, R1_1, R2_1, Rv_1, Wsh1):
  src = edge_index[0].astype(jnp.int32)
  dst = edge_index[1].astype(jnp.int32)
  src_pad = jnp.pad(src, (0, E_PAD - E))
  dst_pad = jnp.pad(dst, (0, E_PAD - E))

  x_pad = jnp.pad(x, ((0, N_PAD - N), (0, 0)))
  pos_pad = jnp.pad(pos, ((0, N_PAD - N), (0, 0)))
  px = pos_pad[:, 0]
  py = pos_pad[:, 1]
  pz = pos_pad[:, 2]
  batch3 = jnp.pad(batch.astype(jnp.int32), (0, N_PAD - N),
                   constant_values=G).reshape(N_PAD // BN, 1, BN)
  bemb2 = b_emb.reshape(1, 64)
  wsh0_16 = jnp.pad(Wsh0, ((0, 7), (0, 0)))
  wsh1_16 = jnp.pad(Wsh1, ((0, 7), (0, 0)))
  zrows = jnp.zeros((N_PAD // 16, TD), jnp.float32)

  rel8 = _sc_geo(px, py, pz, dst_pad, src_pad)

  tdst, tsrc = _node0(x_pad, W_emb, bemb2, Wq0, Wk0, Wv0)
  edst, esrc = _sc_gather(tdst, tsrc, dst_pad, src_pad)
  ev = _edge(edst, esrc, rel8, R1_0, R2_0, Rv_0, wsh0_16)
  part0 = _sc_scatter(ev, dst_pad, zrows)

  tdst, tsrc = _node1(part0, Wq1, Wk1, Wv1)
  edst, esrc = _sc_gather(tdst, tsrc, dst_pad, src_pad)
  ev = _edge(edst, esrc, rel8, R1_1, R2_1, Rv_1, wsh1_16)
  part1 = _sc_scatter(ev, dst_pad, zrows)

  return _pool(part1, batch3)


# speedup vs baseline: 4.1585x; 1.1185x over previous
import functools

import jax
import jax.numpy as jnp
from jax import lax
from jax.experimental import pallas as pl
from jax.experimental.pallas import tpu as pltpu
from jax.experimental.pallas import tpu_sc as plsc

N = 10000
E = 320000
G = 64
DH = 128
NB = 16
RMAX = 2.5
PI = 3.14159265358979

N_PAD = 10240
BN = 1024
C = 128
NW = 32
CHUNKS_PER_W = 79
E_PAD = C * NW * CHUNKS_PER_W
BE = 512
TD = 128
TDT = 128
TST = 256

_HI = lax.Precision.DEFAULT


def _silu(x):
  return x * (1.0 / (1.0 + jnp.exp(-x)))


def _node0_body(x_ref, wemb_ref, bemb_ref, wq_ref, wk_ref, wv_ref,
                tdst_ref, tsrc_ref):
  h = jnp.dot(x_ref[...], wemb_ref[...], precision=_HI,
              preferred_element_type=jnp.float32) + bemb_ref[...]
  q = jnp.dot(h, wq_ref[...], precision=_HI, preferred_element_type=jnp.float32)
  k = jnp.dot(h, wk_ref[...], precision=_HI, preferred_element_type=jnp.float32)
  v = jnp.dot(h, wv_ref[...], precision=_HI, preferred_element_type=jnp.float32)
  tdst_ref[...] = q
  tsrc_ref[...] = jnp.concatenate([k, v], axis=1)


def _node0(x_pad, wemb, bemb, wq, wk, wv):
  grid = N_PAD // BN
  return pl.pallas_call(
      _node0_body,
      grid=(grid,),
      in_specs=[
          pl.BlockSpec((BN, 4), lambda i: (i, 0)),
          pl.BlockSpec((4, 64), lambda i: (0, 0)),
          pl.BlockSpec((1, 64), lambda i: (0, 0)),
          pl.BlockSpec((64, DH), lambda i: (0, 0)),
          pl.BlockSpec((64, DH), lambda i: (0, 0)),
          pl.BlockSpec((64, DH), lambda i: (0, 0)),
      ],
      out_specs=[
          pl.BlockSpec((BN, TDT), lambda i: (i, 0)),
          pl.BlockSpec((BN, TST), lambda i: (i, 0)),
      ],
      out_shape=[
          jax.ShapeDtypeStruct((N_PAD, TDT), jnp.float32),
          jax.ShapeDtypeStruct((N_PAD, TST), jnp.float32),
      ],
  )(x_pad, wemb, bemb, wq, wk, wv)


def _node1_body(p_ref, wq_ref, wk_ref, wv_ref, tdst_ref, tsrc_ref):
  p0 = p_ref[0]
  p1 = p_ref[1]
  den = p0[:, 64:65]
  h = jnp.concatenate([p0[:, :64], p1[:, :64]], axis=1) * (1.0 / (den + 1e-9))
  q = jnp.dot(h, wq_ref[...], precision=_HI, preferred_element_type=jnp.float32)
  k = jnp.dot(h, wk_ref[...], precision=_HI, preferred_element_type=jnp.float32)
  v = jnp.dot(h, wv_ref[...], precision=_HI, preferred_element_type=jnp.float32)
  tdst_ref[...] = q
  tsrc_ref[...] = jnp.concatenate([k, v], axis=1)


def _node1(partials, wq, wk, wv):
  grid = N_PAD // BN
  return pl.pallas_call(
      _node1_body,
      grid=(grid,),
      in_specs=[
          pl.BlockSpec((2, BN, TD), lambda i: (0, i, 0)),
          pl.BlockSpec((DH, DH), lambda i: (0, 0)),
          pl.BlockSpec((DH, DH), lambda i: (0, 0)),
          pl.BlockSpec((DH, DH), lambda i: (0, 0)),
      ],
      out_specs=[
          pl.BlockSpec((BN, TDT), lambda i: (i, 0)),
          pl.BlockSpec((BN, TST), lambda i: (i, 0)),
      ],
      out_shape=[
          jax.ShapeDtypeStruct((N_PAD, TDT), jnp.float32),
          jax.ShapeDtypeStruct((N_PAD, TST), jnp.float32),
      ],
  )(partials, wq, wk, wv)


def _sc_geo_body(px_ref, py_ref, pz_ref, dst_ref, src_ref, rel_ref,
                 pxv, pyv, pzv, idxd, idxs, rx, ry, rz):
  c = lax.axis_index("c")
  s = lax.axis_index("s")
  w = s * 2 + c
  pltpu.sync_copy(px_ref, pxv)
  pltpu.sync_copy(py_ref, pyv)
  pltpu.sync_copy(pz_ref, pzv)

  def body(i, carry):
    off = (i * NW + w) * C
    pltpu.sync_copy(dst_ref.at[pl.ds(off, C)], idxd)
    pltpu.sync_copy(src_ref.at[pl.ds(off, C)], idxs)
    for j in range(C // 16):
      sl = pl.ds(j * 16, 16)
      i_s = idxs[sl]
      i_d = idxd[sl]
      rx[sl] = plsc.load_gather(pxv, [i_s]) - plsc.load_gather(pxv, [i_d])
      ry[sl] = plsc.load_gather(pyv, [i_s]) - plsc.load_gather(pyv, [i_d])
      rz[sl] = plsc.load_gather(pzv, [i_s]) - plsc.load_gather(pzv, [i_d])
    pltpu.sync_copy(rx, rel_ref.at[0, pl.ds(off, C)])
    pltpu.sync_copy(ry, rel_ref.at[1, pl.ds(off, C)])
    pltpu.sync_copy(rz, rel_ref.at[2, pl.ds(off, C)])
    return carry

  lax.fori_loop(0, CHUNKS_PER_W, body, 0)


def _sc_geo(px, py, pz, dst_pad, src_pad):
  mesh = plsc.VectorSubcoreMesh(core_axis_name="c", subcore_axis_name="s")
  f = pl.kernel(
      _sc_geo_body,
      out_type=jax.ShapeDtypeStruct((8, E_PAD), jnp.float32),
      mesh=mesh,
      scratch_types=[
          pltpu.VMEM((N_PAD,), jnp.float32),
          pltpu.VMEM((N_PAD,), jnp.float32),
          pltpu.VMEM((N_PAD,), jnp.float32),
          pltpu.VMEM((C,), jnp.int32),
          pltpu.VMEM((C,), jnp.int32),
          pltpu.VMEM((C,), jnp.float32),
          pltpu.VMEM((C,), jnp.float32),
          pltpu.VMEM((C,), jnp.float32),
      ],
      compiler_params=pltpu.CompilerParams(needs_layout_passes=False),
  )
  return f(px, py, pz, dst_pad, src_pad)


def _sc_gather_body(tdst_ref, tsrc_ref, dst_ref, src_ref, edst_ref, esrc_ref,
                    idxd, idxs, bufd, bufs, wsem):
  c = lax.axis_index("c")
  s = lax.axis_index("s")
  w = s * 2 + c

  def body(i, carry):
    b = i % 2
    off = (i * NW + w) * C

    @pl.when(i >= 2)
    def _():
      pltpu.make_async_copy(bufd.at[b], edst_ref.at[pl.ds(0, C)],
                            wsem.at[b]).wait()
      pltpu.make_async_copy(bufs.at[b], esrc_ref.at[pl.ds(0, C)],
                            wsem.at[b]).wait()

    pltpu.sync_copy(dst_ref.at[pl.ds(off, C)], idxd)
    pltpu.sync_copy(src_ref.at[pl.ds(off, C)], idxs)
    pltpu.sync_copy(tdst_ref.at[idxd], bufd.at[b])
    pltpu.sync_copy(tsrc_ref.at[idxs], bufs.at[b])
    pltpu.make_async_copy(bufd.at[b], edst_ref.at[pl.ds(off, C)],
                          wsem.at[b]).start()
    pltpu.make_async_copy(bufs.at[b], esrc_ref.at[pl.ds(off, C)],
                          wsem.at[b]).start()
    return carry

  lax.fori_loop(0, CHUNKS_PER_W, body, 0)
  for b in (0, 1):
    pltpu.make_async_copy(bufd.at[b], edst_ref.at[pl.ds(0, C)],
                          wsem.at[b]).wait()
    pltpu.make_async_copy(bufs.at[b], esrc_ref.at[pl.ds(0, C)],
                          wsem.at[b]).wait()


def _sc_gather(tdst, tsrc, dst_pad, src_pad):
  mesh = plsc.VectorSubcoreMesh(core_axis_name="c", subcore_axis_name="s")
  f = pl.kernel(
      _sc_gather_body,
      out_type=[
          jax.ShapeDtypeStruct((E_PAD, TDT), jnp.float32),
          jax.ShapeDtypeStruct((E_PAD, TST), jnp.float32),
      ],
      mesh=mesh,
      scratch_types=[
          pltpu.VMEM((C,), jnp.int32),
          pltpu.VMEM((C,), jnp.int32),
          pltpu.VMEM((2, C, TDT), jnp.float32),
          pltpu.VMEM((2, C, TST), jnp.float32),
          pltpu.SemaphoreType.DMA((2,)),
      ],
  )
  return f(tdst, tsrc, dst_pad, src_pad)


def _edge_body(ed_ref, es_ref, rel_ref, r1_ref, r2_ref, rv_ref, wsh_ref,
               out_ref):
  i = pl.program_id(0)

  @pl.when(i >= E // BE)
  def _():
    out_ref[...] = jnp.zeros_like(out_ref)

  @pl.when(i < E // BE)
  def _():
    q = ed_ref[...]
    es = es_ref[...]
    k0 = es[:, :128]
    v0 = es[:, 128:256]

    relT = rel_ref[...]
    rx = relT[0:1, :]
    ry = relT[1:2, :]
    rz = relT[2:3, :]
    r2T = rx * rx + ry * ry + rz * rz + 1e-12
    rT = jnp.sqrt(r2T)
    inv_r = 1.0 / (rT + 1e-9)
    dx = rx * inv_r
    dy = ry * inv_r
    dz = rz * inv_r

    centers = lax.broadcasted_iota(jnp.int32, (NB, BE), 0).astype(
        jnp.float32) * (RMAX / (NB - 1))
    width = RMAX / NB
    tT = (jnp.broadcast_to(rT, (NB, BE)) - centers) * (1.0 / width)
    rbfT = jnp.exp(-(tT * tT))
    envT = jnp.where(rT < RMAX, 0.5 * (jnp.cos(PI / RMAX * rT) + 1.0), 0.0)
    rbT = rbfT * envT
    rb = jnp.transpose(rbT)

    hidden = _silu(jnp.dot(rb, r1_ref[...], precision=_HI,
                           preferred_element_type=jnp.float32))
    rk = jnp.dot(hidden, r2_ref[...], precision=_HI,
                 preferred_element_type=jnp.float32)
    rvv = jnp.dot(hidden, rv_ref[...], precision=_HI,
                  preferred_element_type=jnp.float32)

    one = jnp.ones_like(dx)
    shT = jnp.concatenate([
        one, dx, dy, dz,
        1.7320508 * dx * dy, 1.7320508 * dy * dz,
        0.5 * (3.0 * dz * dz - 1.0),
        1.7320508 * dx * dz, 0.8660254 * (dx * dx - dy * dy),
        jnp.zeros((7, BE), jnp.float32),
    ], axis=0)
    sh = jnp.transpose(shT)
    shw = jnp.dot(sh, wsh_ref[...], precision=_HI,
                  preferred_element_type=jnp.float32)

    k = k0 * rk + shw
    v = v0 * rvv
    logits = jnp.sum(q * k, axis=1, keepdims=True) * (DH ** -0.5)
    ex = jnp.exp(logits)

    zpad = jnp.zeros((BE, 63), jnp.float32)
    out_ref[0] = jnp.concatenate([ex * v[:, :64], ex, zpad], axis=1)
    out_ref[1] = jnp.concatenate([ex * v[:, 64:], ex, zpad], axis=1)


def _edge(edst, esrc, rel8, r1, r2, rv, wsh16):
  grid = E_PAD // BE
  return pl.pallas_call(
      _edge_body,
      grid=(grid,),
      in_specs=[
          pl.BlockSpec((BE, TDT), lambda i: (i, 0)),
          pl.BlockSpec((BE, TST), lambda i: (i, 0)),
          pl.BlockSpec((8, BE), lambda i: (0, i)),
          pl.BlockSpec((NB, 64), lambda i: (0, 0)),
          pl.BlockSpec((64, DH), lambda i: (0, 0)),
          pl.BlockSpec((64, DH), lambda i: (0, 0)),
          pl.BlockSpec((16, DH), lambda i: (0, 0)),
      ],
      out_specs=pl.BlockSpec((2, BE, TD), lambda i: (0, i, 0)),
      out_shape=jax.ShapeDtypeStruct((2, E_PAD, TD), jnp.float32),
  )(edst, esrc, rel8, r1, r2, rv, wsh16)


def _sc_scatter_body(ev_ref, dst_ref, zrows_ref, out_ref, idxb, buf, acc,
                     asem):
  c = lax.axis_index("c")
  s = lax.axis_index("s")
  rpt = N_PAD // 16
  base = s * rpt
  pltpu.sync_copy(zrows_ref.at[pl.ds(0, rpt)], acc.at[pl.ds(base, rpt)])
  plsc.subcore_barrier()

  def body(i, carry):
    b = i % 2
    off = (i * 16 + s) * C

    @pl.when(i >= 2)
    def _():
      pltpu.make_async_copy(buf.at[b], acc.at[idxb.at[b]],
                            asem.at[b]).wait()

    pltpu.sync_copy(dst_ref.at[pl.ds(off, C)], idxb.at[b])
    pltpu.sync_copy(ev_ref.at[c, pl.ds(off, C)], buf.at[b])
    pltpu.make_async_copy(buf.at[b], acc.at[idxb.at[b]],
                          asem.at[b]).start(add=True)
    return carry

  lax.fori_loop(0, CHUNKS_PER_W * 2, body, 0)
  for b in (0, 1):
    pltpu.make_async_copy(buf.at[b], acc.at[idxb.at[b]], asem.at[b]).wait()
  plsc.subcore_barrier()
  pltpu.sync_copy(acc.at[pl.ds(base, rpt)], out_ref.at[c, pl.ds(base, rpt)])


def _sc_scatter(ev, dst_pad, zrows):
  mesh = plsc.VectorSubcoreMesh(core_axis_name="c", subcore_axis_name="s")
  f = pl.kernel(
      _sc_scatter_body,
      out_type=jax.ShapeDtypeStruct((2, N_PAD, TD), jnp.float32),
      mesh=mesh,
      scratch_types=[
          pltpu.VMEM((2, C), jnp.int32),
          pltpu.VMEM((2, C, TD), jnp.float32),
          pltpu.VMEM_SHARED((N_PAD, TD), jnp.float32),
          pltpu.SemaphoreType.DMA((2,)),
      ],
  )
  return f(ev, dst_pad, zrows)


def _pool_body(p_ref, batch_ref, out_ref, acc):
  i = pl.program_id(0)

  @pl.when(i == 0)
  def _():
    acc[...] = jnp.zeros_like(acc)

  p0 = p_ref[0]
  p1 = p_ref[1]
  den = p0[:, 64:65]
  h = jnp.concatenate([p0[:, :64], p1[:, :64]], axis=1) * (1.0 / (den + 1e-9))

  bt = batch_ref[0]
  oh = (lax.broadcasted_iota(jnp.int32, (G, BN), 0) == bt).astype(jnp.float32)
  hext = jnp.concatenate(
      [h, jnp.ones((BN, 1), jnp.float32), jnp.zeros((BN, 127), jnp.float32)],
      axis=1)
  acc[...] += jnp.dot(oh, hext, precision=_HI,
                      preferred_element_type=jnp.float32)

  @pl.when(i == (N_PAD // BN) - 1)
  def _():
    cnt = acc[:, 128:129]
    out_ref[...] = acc[:, :128] * (1.0 / jnp.maximum(cnt, 1.0))


def _pool(partials, batch3):
  grid = N_PAD // BN
  return pl.pallas_call(
      _pool_body,
      grid=(grid,),
      in_specs=[
          pl.BlockSpec((2, BN, TD), lambda i: (0, i, 0)),
          pl.BlockSpec((1, 1, BN), lambda i: (i, 0, 0)),
      ],
      out_specs=pl.BlockSpec((G, DH), lambda i: (0, 0)),
      out_shape=jax.ShapeDtypeStruct((G, DH), jnp.float32),
      scratch_shapes=[pltpu.VMEM((G, 256), jnp.float32)],
      compiler_params=pltpu.CompilerParams(
          dimension_semantics=("arbitrary",)),
  )(partials, batch3)


def kernel(x, pos, edge_index, batch, W_emb, b_emb,
           Wq0, Wk0, Wv0, R1_0, R2_0, Rv_0, Wsh0,
           Wq1, Wk1, Wv1, R1_1, R2_1, Rv_1, Wsh1):
  src = edge_index[0].astype(jnp.int32)
  dst = edge_index[1].astype(jnp.int32)
  src_pad = jnp.pad(src, (0, E_PAD - E))
  dst_pad = jnp.pad(dst, (0, E_PAD - E))

  x_pad = jnp.pad(x, ((0, N_PAD - N), (0, 0)))
  pos_pad = jnp.pad(pos, ((0, N_PAD - N), (0, 0)))
  px = pos_pad[:, 0]
  py = pos_pad[:, 1]
  pz = pos_pad[:, 2]
  batch3 = jnp.pad(batch.astype(jnp.int32), (0, N_PAD - N),
                   constant_values=G).reshape(N_PAD // BN, 1, BN)
  bemb2 = b_emb.reshape(1, 64)
  wsh0_16 = jnp.pad(Wsh0, ((0, 7), (0, 0)))
  wsh1_16 = jnp.pad(Wsh1, ((0, 7), (0, 0)))
  zrows = jnp.zeros((N_PAD // 16, TD), jnp.float32)

  rel8 = _sc_geo(px, py, pz, dst_pad, src_pad)

  tdst, tsrc = _node0(x_pad, W_emb, bemb2, Wq0, Wk0, Wv0)
  edst, esrc = _sc_gather(tdst, tsrc, dst_pad, src_pad)
  ev = _edge(edst, esrc, rel8, R1_0, R2_0, Rv_0, wsh0_16)
  part0 = _sc_scatter(ev, dst_pad, zrows)

  tdst, tsrc = _node1(part0, Wq1, Wk1, Wv1)
  edst, esrc = _sc_gather(tdst, tsrc, dst_pad, src_pad)
  ev = _edge(edst, esrc, rel8, R1_1, R2_1, Rv_1, wsh1_16)
  part1 = _sc_scatter(ev, dst_pad, zrows)

  return _pool(part1, batch3)


# speedup vs baseline: 4.4481x; 1.0696x over previous
import functools

import jax
import jax.numpy as jnp
from jax import lax
from jax.experimental import pallas as pl
from jax.experimental.pallas import tpu as pltpu
from jax.experimental.pallas import tpu_sc as plsc

N = 10000
E = 320000
G = 64
DH = 128
NB = 16
RMAX = 2.5
PI = 3.14159265358979

N_PAD = 10240
BN = 1024
C = 128
NW = 32
CHUNKS_PER_W = 79
E_PAD = C * NW * CHUNKS_PER_W
BE = 512
TD = 128
TDT = 128
TST = 256

_HI = lax.Precision.DEFAULT


def _silu(x):
  return x * (1.0 / (1.0 + jnp.exp(-x)))


def _node0_body(x_ref, wemb_ref, bemb_ref, wq_ref, wk_ref, wv_ref,
                tdst_ref, tsrc_ref):
  h = jnp.dot(x_ref[...], wemb_ref[...], precision=_HI,
              preferred_element_type=jnp.float32) + bemb_ref[...]
  q = jnp.dot(h, wq_ref[...], precision=_HI, preferred_element_type=jnp.float32)
  k = jnp.dot(h, wk_ref[...], precision=_HI, preferred_element_type=jnp.float32)
  v = jnp.dot(h, wv_ref[...], precision=_HI, preferred_element_type=jnp.float32)
  kb = lax.bitcast_convert_type(k.astype(jnp.bfloat16), jnp.uint16)
  vb = lax.bitcast_convert_type(v.astype(jnp.bfloat16), jnp.uint16)
  word = kb.astype(jnp.uint32) | (vb.astype(jnp.uint32) << 16)
  tdst_ref[...] = q
  tsrc_ref[...] = lax.bitcast_convert_type(word, jnp.int32)


def _node0(x_pad, wemb, bemb, wq, wk, wv):
  grid = N_PAD // BN
  return pl.pallas_call(
      _node0_body,
      grid=(grid,),
      in_specs=[
          pl.BlockSpec((BN, 4), lambda i: (i, 0)),
          pl.BlockSpec((4, 64), lambda i: (0, 0)),
          pl.BlockSpec((1, 64), lambda i: (0, 0)),
          pl.BlockSpec((64, DH), lambda i: (0, 0)),
          pl.BlockSpec((64, DH), lambda i: (0, 0)),
          pl.BlockSpec((64, DH), lambda i: (0, 0)),
      ],
      out_specs=[
          pl.BlockSpec((BN, TDT), lambda i: (i, 0)),
          pl.BlockSpec((BN, TDT), lambda i: (i, 0)),
      ],
      out_shape=[
          jax.ShapeDtypeStruct((N_PAD, TDT), jnp.float32),
          jax.ShapeDtypeStruct((N_PAD, TDT), jnp.int32),
      ],
  )(x_pad, wemb, bemb, wq, wk, wv)


def _node1_body(p_ref, wq_ref, wk_ref, wv_ref, tdst_ref, tsrc_ref):
  p0 = p_ref[0]
  p1 = p_ref[1]
  den = p0[:, 64:65]
  h = jnp.concatenate([p0[:, :64], p1[:, :64]], axis=1) * (1.0 / (den + 1e-9))
  q = jnp.dot(h, wq_ref[...], precision=_HI, preferred_element_type=jnp.float32)
  k = jnp.dot(h, wk_ref[...], precision=_HI, preferred_element_type=jnp.float32)
  v = jnp.dot(h, wv_ref[...], precision=_HI, preferred_element_type=jnp.float32)
  kb = lax.bitcast_convert_type(k.astype(jnp.bfloat16), jnp.uint16)
  vb = lax.bitcast_convert_type(v.astype(jnp.bfloat16), jnp.uint16)
  word = kb.astype(jnp.uint32) | (vb.astype(jnp.uint32) << 16)
  tdst_ref[...] = q
  tsrc_ref[...] = lax.bitcast_convert_type(word, jnp.int32)


def _node1(partials, wq, wk, wv):
  grid = N_PAD // BN
  return pl.pallas_call(
      _node1_body,
      grid=(grid,),
      in_specs=[
          pl.BlockSpec((2, BN, TD), lambda i: (0, i, 0)),
          pl.BlockSpec((DH, DH), lambda i: (0, 0)),
          pl.BlockSpec((DH, DH), lambda i: (0, 0)),
          pl.BlockSpec((DH, DH), lambda i: (0, 0)),
      ],
      out_specs=[
          pl.BlockSpec((BN, TDT), lambda i: (i, 0)),
          pl.BlockSpec((BN, TDT), lambda i: (i, 0)),
      ],
      out_shape=[
          jax.ShapeDtypeStruct((N_PAD, TDT), jnp.float32),
          jax.ShapeDtypeStruct((N_PAD, TDT), jnp.int32),
      ],
  )(partials, wq, wk, wv)


def _sc_geo_body(px_ref, py_ref, pz_ref, dst_ref, src_ref, rel_ref,
                 pxv, pyv, pzv, idxd, idxs, rx, ry, rz):
  c = lax.axis_index("c")
  s = lax.axis_index("s")
  w = s * 2 + c
  pltpu.sync_copy(px_ref, pxv)
  pltpu.sync_copy(py_ref, pyv)
  pltpu.sync_copy(pz_ref, pzv)

  def body(i, carry):
    off = (i * NW + w) * C
    pltpu.sync_copy(dst_ref.at[pl.ds(off, C)], idxd)
    pltpu.sync_copy(src_ref.at[pl.ds(off, C)], idxs)
    for j in range(C // 16):
      sl = pl.ds(j * 16, 16)
      i_s = idxs[sl]
      i_d = idxd[sl]
      rx[sl] = plsc.load_gather(pxv, [i_s]) - plsc.load_gather(pxv, [i_d])
      ry[sl] = plsc.load_gather(pyv, [i_s]) - plsc.load_gather(pyv, [i_d])
      rz[sl] = plsc.load_gather(pzv, [i_s]) - plsc.load_gather(pzv, [i_d])
    pltpu.sync_copy(rx, rel_ref.at[0, pl.ds(off, C)])
    pltpu.sync_copy(ry, rel_ref.at[1, pl.ds(off, C)])
    pltpu.sync_copy(rz, rel_ref.at[2, pl.ds(off, C)])
    return carry

  lax.fori_loop(0, CHUNKS_PER_W, body, 0)


def _sc_geo(px, py, pz, dst_pad, src_pad):
  mesh = plsc.VectorSubcoreMesh(core_axis_name="c", subcore_axis_name="s")
  f = pl.kernel(
      _sc_geo_body,
      out_type=jax.ShapeDtypeStruct((8, E_PAD), jnp.float32),
      mesh=mesh,
      scratch_types=[
          pltpu.VMEM((N_PAD,), jnp.float32),
          pltpu.VMEM((N_PAD,), jnp.float32),
          pltpu.VMEM((N_PAD,), jnp.float32),
          pltpu.VMEM((C,), jnp.int32),
          pltpu.VMEM((C,), jnp.int32),
          pltpu.VMEM((C,), jnp.float32),
          pltpu.VMEM((C,), jnp.float32),
          pltpu.VMEM((C,), jnp.float32),
      ],
      compiler_params=pltpu.CompilerParams(needs_layout_passes=False),
  )
  return f(px, py, pz, dst_pad, src_pad)


def _sc_gather_body(tdst_ref, tsrc_ref, dst_ref, src_ref, edst_ref, esrc_ref,
                    idxd, idxs, bufd, bufs, wsem):
  c = lax.axis_index("c")
  s = lax.axis_index("s")
  w = s * 2 + c

  def body(i, carry):
    b = i % 2
    off = (i * NW + w) * C

    @pl.when(i >= 2)
    def _():
      pltpu.make_async_copy(bufd.at[b], edst_ref.at[pl.ds(0, C)],
                            wsem.at[b]).wait()
      pltpu.make_async_copy(bufs.at[b], esrc_ref.at[pl.ds(0, C)],
                            wsem.at[b]).wait()

    pltpu.sync_copy(dst_ref.at[pl.ds(off, C)], idxd)
    pltpu.sync_copy(src_ref.at[pl.ds(off, C)], idxs)
    pltpu.sync_copy(tdst_ref.at[idxd], bufd.at[b])
    pltpu.sync_copy(tsrc_ref.at[idxs], bufs.at[b])
    pltpu.make_async_copy(bufd.at[b], edst_ref.at[pl.ds(off, C)],
                          wsem.at[b]).start()
    pltpu.make_async_copy(bufs.at[b], esrc_ref.at[pl.ds(off, C)],
                          wsem.at[b]).start()
    return carry

  lax.fori_loop(0, CHUNKS_PER_W, body, 0)
  for b in (0, 1):
    pltpu.make_async_copy(bufd.at[b], edst_ref.at[pl.ds(0, C)],
                          wsem.at[b]).wait()
    pltpu.make_async_copy(bufs.at[b], esrc_ref.at[pl.ds(0, C)],
                          wsem.at[b]).wait()


def _sc_gather(tdst, tsrc, dst_pad, src_pad):
  mesh = plsc.VectorSubcoreMesh(core_axis_name="c", subcore_axis_name="s")
  f = pl.kernel(
      _sc_gather_body,
      out_type=[
          jax.ShapeDtypeStruct((E_PAD, TDT), jnp.float32),
          jax.ShapeDtypeStruct((E_PAD, TDT), jnp.int32),
      ],
      mesh=mesh,
      scratch_types=[
          pltpu.VMEM((C,), jnp.int32),
          pltpu.VMEM((C,), jnp.int32),
          pltpu.VMEM((2, C, TDT), jnp.float32),
          pltpu.VMEM((2, C, TDT), jnp.int32),
          pltpu.SemaphoreType.DMA((2,)),
      ],
  )
  return f(tdst, tsrc, dst_pad, src_pad)


def _edge_body(ed_ref, es_ref, rel_ref, r1_ref, r2_ref, rv_ref, wsh_ref,
               out_ref):
  i = pl.program_id(0)

  @pl.when(i >= E // BE)
  def _():
    out_ref[...] = jnp.zeros_like(out_ref)

  @pl.when(i < E // BE)
  def _():
    q = ed_ref[...]
    wu = lax.bitcast_convert_type(es_ref[...], jnp.uint32)
    k0 = lax.bitcast_convert_type(
        (wu & 0xFFFF).astype(jnp.uint16), jnp.bfloat16).astype(jnp.float32)
    v0 = lax.bitcast_convert_type(
        (wu >> 16).astype(jnp.uint16), jnp.bfloat16).astype(jnp.float32)

    relT = rel_ref[...]
    rx = relT[0:1, :]
    ry = relT[1:2, :]
    rz = relT[2:3, :]
    r2T = rx * rx + ry * ry + rz * rz + 1e-12
    rT = jnp.sqrt(r2T)
    inv_r = 1.0 / (rT + 1e-9)
    dx = rx * inv_r
    dy = ry * inv_r
    dz = rz * inv_r

    centers = lax.broadcasted_iota(jnp.int32, (NB, BE), 0).astype(
        jnp.float32) * (RMAX / (NB - 1))
    width = RMAX / NB
    tT = (jnp.broadcast_to(rT, (NB, BE)) - centers) * (1.0 / width)
    rbfT = jnp.exp(-(tT * tT))
    envT = jnp.where(rT < RMAX, 0.5 * (jnp.cos(PI / RMAX * rT) + 1.0), 0.0)
    rbT = rbfT * envT
    rb = jnp.transpose(rbT)

    hidden = _silu(jnp.dot(rb, r1_ref[...], precision=_HI,
                           preferred_element_type=jnp.float32))
    rk = jnp.dot(hidden, r2_ref[...], precision=_HI,
                 preferred_element_type=jnp.float32)
    rvv = jnp.dot(hidden, rv_ref[...], precision=_HI,
                  preferred_element_type=jnp.float32)

    one = jnp.ones_like(dx)
    shT = jnp.concatenate([
        one, dx, dy, dz,
        1.7320508 * dx * dy, 1.7320508 * dy * dz,
        0.5 * (3.0 * dz * dz - 1.0),
        1.7320508 * dx * dz, 0.8660254 * (dx * dx - dy * dy),
        jnp.zeros((7, BE), jnp.float32),
    ], axis=0)
    sh = jnp.transpose(shT)
    shw = jnp.dot(sh, wsh_ref[...], precision=_HI,
                  preferred_element_type=jnp.float32)

    k = k0 * rk + shw
    v = v0 * rvv
    logits = jnp.sum(q * k, axis=1, keepdims=True) * (DH ** -0.5)
    ex = jnp.exp(logits)

    zpad = jnp.zeros((BE, 63), jnp.float32)
    out_ref[0] = jnp.concatenate([ex * v[:, :64], ex, zpad], axis=1)
    out_ref[1] = jnp.concatenate([ex * v[:, 64:], ex, zpad], axis=1)


def _edge(edst, esrc, rel8, r1, r2, rv, wsh16):
  grid = E_PAD // BE
  return pl.pallas_call(
      _edge_body,
      grid=(grid,),
      in_specs=[
          pl.BlockSpec((BE, TDT), lambda i: (i, 0)),
          pl.BlockSpec((BE, TDT), lambda i: (i, 0)),
          pl.BlockSpec((8, BE), lambda i: (0, i)),
          pl.BlockSpec((NB, 64), lambda i: (0, 0)),
          pl.BlockSpec((64, DH), lambda i: (0, 0)),
          pl.BlockSpec((64, DH), lambda i: (0, 0)),
          pl.BlockSpec((16, DH), lambda i: (0, 0)),
      ],
      out_specs=pl.BlockSpec((2, BE, TD), lambda i: (0, i, 0)),
      out_shape=jax.ShapeDtypeStruct((2, E_PAD, TD), jnp.float32),
  )(edst, esrc, rel8, r1, r2, rv, wsh16)


def _sc_scatter_body(ev_ref, dst_ref, zrows_ref, out_ref, idxb, buf, acc,
                     asem):
  c = lax.axis_index("c")
  s = lax.axis_index("s")
  rpt = N_PAD // 16
  base = s * rpt
  pltpu.sync_copy(zrows_ref.at[pl.ds(0, rpt)], acc.at[pl.ds(base, rpt)])
  plsc.subcore_barrier()

  def body(i, carry):
    b = i % 2
    off = (i * 16 + s) * C

    @pl.when(i >= 2)
    def _():
      pltpu.make_async_copy(buf.at[b], acc.at[idxb.at[b]],
                            asem.at[b]).wait()

    pltpu.sync_copy(dst_ref.at[pl.ds(off, C)], idxb.at[b])
    pltpu.sync_copy(ev_ref.at[c, pl.ds(off, C)], buf.at[b])
    pltpu.make_async_copy(buf.at[b], acc.at[idxb.at[b]],
                          asem.at[b]).start(add=True)
    return carry

  lax.fori_loop(0, CHUNKS_PER_W * 2, body, 0)
  for b in (0, 1):
    pltpu.make_async_copy(buf.at[b], acc.at[idxb.at[b]], asem.at[b]).wait()
  plsc.subcore_barrier()
  pltpu.sync_copy(acc.at[pl.ds(base, rpt)], out_ref.at[c, pl.ds(base, rpt)])


def _sc_scatter(ev, dst_pad, zrows):
  mesh = plsc.VectorSubcoreMesh(core_axis_name="c", subcore_axis_name="s")
  f = pl.kernel(
      _sc_scatter_body,
      out_type=jax.ShapeDtypeStruct((2, N_PAD, TD), jnp.float32),
      mesh=mesh,
      scratch_types=[
          pltpu.VMEM((2, C), jnp.int32),
          pltpu.VMEM((2, C, TD), jnp.float32),
          pltpu.VMEM_SHARED((N_PAD, TD), jnp.float32),
          pltpu.SemaphoreType.DMA((2,)),
      ],
  )
  return f(ev, dst_pad, zrows)


def _pool_body(p_ref, batch_ref, out_ref, acc):
  i = pl.program_id(0)

  @pl.when(i == 0)
  def _():
    acc[...] = jnp.zeros_like(acc)

  p0 = p_ref[0]
  p1 = p_ref[1]
  den = p0[:, 64:65]
  h = jnp.concatenate([p0[:, :64], p1[:, :64]], axis=1) * (1.0 / (den + 1e-9))

  bt = batch_ref[0]
  oh = (lax.broadcasted_iota(jnp.int32, (G, BN), 0) == bt).astype(jnp.float32)
  hext = jnp.concatenate(
      [h, jnp.ones((BN, 1), jnp.float32), jnp.zeros((BN, 127), jnp.float32)],
      axis=1)
  acc[...] += jnp.dot(oh, hext, precision=_HI,
                      preferred_element_type=jnp.float32)

  @pl.when(i == (N_PAD // BN) - 1)
  def _():
    cnt = acc[:, 128:129]
    out_ref[...] = acc[:, :128] * (1.0 / jnp.maximum(cnt, 1.0))


def _pool(partials, batch3):
  grid = N_PAD // BN
  return pl.pallas_call(
      _pool_body,
      grid=(grid,),
      in_specs=[
          pl.BlockSpec((2, BN, TD), lambda i: (0, i, 0)),
          pl.BlockSpec((1, 1, BN), lambda i: (i, 0, 0)),
      ],
      out_specs=pl.BlockSpec((G, DH), lambda i: (0, 0)),
      out_shape=jax.ShapeDtypeStruct((G, DH), jnp.float32),
      scratch_shapes=[pltpu.VMEM((G, 256), jnp.float32)],
      compiler_params=pltpu.CompilerParams(
          dimension_semantics=("arbitrary",)),
  )(partials, batch3)


def kernel(x, pos, edge_index, batch, W_emb, b_emb,
           Wq0, Wk0, Wv0, R1_0, R2_0, Rv_0, Wsh0,
           Wq1, Wk1, Wv1, R1_1, R2_1, Rv_1, Wsh1):
  src = edge_index[0].astype(jnp.int32)
  dst = edge_index[1].astype(jnp.int32)
  src_pad = jnp.pad(src, (0, E_PAD - E))
  dst_pad = jnp.pad(dst, (0, E_PAD - E))

  x_pad = jnp.pad(x, ((0, N_PAD - N), (0, 0)))
  pos_pad = jnp.pad(pos, ((0, N_PAD - N), (0, 0)))
  px = pos_pad[:, 0]
  py = pos_pad[:, 1]
  pz = pos_pad[:, 2]
  batch3 = jnp.pad(batch.astype(jnp.int32), (0, N_PAD - N),
                   constant_values=G).reshape(N_PAD // BN, 1, BN)
  bemb2 = b_emb.reshape(1, 64)
  wsh0_16 = jnp.pad(Wsh0, ((0, 7), (0, 0)))
  wsh1_16 = jnp.pad(Wsh1, ((0, 7), (0, 0)))
  zrows = jnp.zeros((N_PAD // 16, TD), jnp.float32)

  rel8 = _sc_geo(px, py, pz, dst_pad, src_pad)

  tdst, tsrc = _node0(x_pad, W_emb, bemb2, Wq0, Wk0, Wv0)
  edst, esrc = _sc_gather(tdst, tsrc, dst_pad, src_pad)
  ev = _edge(edst, esrc, rel8, R1_0, R2_0, Rv_0, wsh0_16)
  part0 = _sc_scatter(ev, dst_pad, zrows)

  tdst, tsrc = _node1(part0, Wq1, Wk1, Wv1)
  edst, esrc = _sc_gather(tdst, tsrc, dst_pad, src_pad)
  ev = _edge(edst, esrc, rel8, R1_1, R2_1, Rv_1, wsh1_16)
  part1 = _sc_scatter(ev, dst_pad, zrows)

  return _pool(part1, batch3)


# speedup vs baseline: 5.2114x; 1.1716x over previous
import functools

import jax
import jax.numpy as jnp
from jax import lax
from jax.experimental import pallas as pl
from jax.experimental.pallas import tpu as pltpu
from jax.experimental.pallas import tpu_sc as plsc

N = 10000
E = 320000
G = 64
DH = 128
NB = 16
RMAX = 2.5
PI = 3.14159265358979

N_PAD = 10240
BN = 1024
C = 128
NW = 32
CHUNKS_PER_W = 79
E_PAD = C * NW * CHUNKS_PER_W
BE = 512
TD = 128
TDT = 128
TST = 256

_HI = lax.Precision.DEFAULT


def _silu(x):
  return x * (1.0 / (1.0 + jnp.exp(-x)))


def _node0_body(x_ref, wemb_ref, bemb_ref, wq_ref, wk_ref, wv_ref,
                tdst_ref, tsrc_ref):
  h = jnp.dot(x_ref[...], wemb_ref[...], precision=_HI,
              preferred_element_type=jnp.float32) + bemb_ref[...]
  q = jnp.dot(h, wq_ref[...], precision=_HI, preferred_element_type=jnp.float32)
  k = jnp.dot(h, wk_ref[...], precision=_HI, preferred_element_type=jnp.float32)
  v = jnp.dot(h, wv_ref[...], precision=_HI, preferred_element_type=jnp.float32)
  kb = lax.bitcast_convert_type(k.astype(jnp.bfloat16), jnp.uint16)
  vb = lax.bitcast_convert_type(v.astype(jnp.bfloat16), jnp.uint16)
  word = kb.astype(jnp.uint32) | (vb.astype(jnp.uint32) << 16)
  tdst_ref[...] = q
  tsrc_ref[...] = lax.bitcast_convert_type(word, jnp.int32)


def _node0(x_pad, wemb, bemb, wq, wk, wv):
  grid = N_PAD // BN
  return pl.pallas_call(
      _node0_body,
      grid=(grid,),
      in_specs=[
          pl.BlockSpec((BN, 4), lambda i: (i, 0)),
          pl.BlockSpec((4, 64), lambda i: (0, 0)),
          pl.BlockSpec((1, 64), lambda i: (0, 0)),
          pl.BlockSpec((64, DH), lambda i: (0, 0)),
          pl.BlockSpec((64, DH), lambda i: (0, 0)),
          pl.BlockSpec((64, DH), lambda i: (0, 0)),
      ],
      out_specs=[
          pl.BlockSpec((BN, TDT), lambda i: (i, 0)),
          pl.BlockSpec((BN, TDT), lambda i: (i, 0)),
      ],
      out_shape=[
          jax.ShapeDtypeStruct((N_PAD, TDT), jnp.float32),
          jax.ShapeDtypeStruct((N_PAD, TDT), jnp.int32),
      ],
  )(x_pad, wemb, bemb, wq, wk, wv)


def _node1_body(p_ref, wq_ref, wk_ref, wv_ref, tdst_ref, tsrc_ref):
  p0 = p_ref[0]
  p1 = p_ref[1]
  den = p0[:, 64:65]
  h = jnp.concatenate([p0[:, :64], p1[:, :64]], axis=1) * (1.0 / (den + 1e-9))
  q = jnp.dot(h, wq_ref[...], precision=_HI, preferred_element_type=jnp.float32)
  k = jnp.dot(h, wk_ref[...], precision=_HI, preferred_element_type=jnp.float32)
  v = jnp.dot(h, wv_ref[...], precision=_HI, preferred_element_type=jnp.float32)
  kb = lax.bitcast_convert_type(k.astype(jnp.bfloat16), jnp.uint16)
  vb = lax.bitcast_convert_type(v.astype(jnp.bfloat16), jnp.uint16)
  word = kb.astype(jnp.uint32) | (vb.astype(jnp.uint32) << 16)
  tdst_ref[...] = q
  tsrc_ref[...] = lax.bitcast_convert_type(word, jnp.int32)


def _node1(partials, wq, wk, wv):
  grid = N_PAD // BN
  return pl.pallas_call(
      _node1_body,
      grid=(grid,),
      in_specs=[
          pl.BlockSpec((2, BN, TD), lambda i: (0, i, 0)),
          pl.BlockSpec((DH, DH), lambda i: (0, 0)),
          pl.BlockSpec((DH, DH), lambda i: (0, 0)),
          pl.BlockSpec((DH, DH), lambda i: (0, 0)),
      ],
      out_specs=[
          pl.BlockSpec((BN, TDT), lambda i: (i, 0)),
          pl.BlockSpec((BN, TDT), lambda i: (i, 0)),
      ],
      out_shape=[
          jax.ShapeDtypeStruct((N_PAD, TDT), jnp.float32),
          jax.ShapeDtypeStruct((N_PAD, TDT), jnp.int32),
      ],
  )(partials, wq, wk, wv)


def _sc_geo_body(px_ref, py_ref, pz_ref, dst_ref, src_ref, rel_ref,
                 pxv, pyv, pzv, idxd, idxs, rx, ry, rz):
  c = lax.axis_index("c")
  s = lax.axis_index("s")
  w = s * 2 + c
  pltpu.sync_copy(px_ref, pxv)
  pltpu.sync_copy(py_ref, pyv)
  pltpu.sync_copy(pz_ref, pzv)

  def body(i, carry):
    off = (i * NW + w) * C
    pltpu.sync_copy(dst_ref.at[pl.ds(off, C)], idxd)
    pltpu.sync_copy(src_ref.at[pl.ds(off, C)], idxs)
    for j in range(C // 16):
      sl = pl.ds(j * 16, 16)
      i_s = idxs[sl]
      i_d = idxd[sl]
      rx[sl] = plsc.load_gather(pxv, [i_s]) - plsc.load_gather(pxv, [i_d])
      ry[sl] = plsc.load_gather(pyv, [i_s]) - plsc.load_gather(pyv, [i_d])
      rz[sl] = plsc.load_gather(pzv, [i_s]) - plsc.load_gather(pzv, [i_d])
    pltpu.sync_copy(rx, rel_ref.at[0, pl.ds(off, C)])
    pltpu.sync_copy(ry, rel_ref.at[1, pl.ds(off, C)])
    pltpu.sync_copy(rz, rel_ref.at[2, pl.ds(off, C)])
    return carry

  lax.fori_loop(0, CHUNKS_PER_W, body, 0)


def _sc_geo(px, py, pz, dst_pad, src_pad):
  mesh = plsc.VectorSubcoreMesh(core_axis_name="c", subcore_axis_name="s")
  f = pl.kernel(
      _sc_geo_body,
      out_type=jax.ShapeDtypeStruct((8, E_PAD), jnp.float32),
      mesh=mesh,
      scratch_types=[
          pltpu.VMEM((N_PAD,), jnp.float32),
          pltpu.VMEM((N_PAD,), jnp.float32),
          pltpu.VMEM((N_PAD,), jnp.float32),
          pltpu.VMEM((C,), jnp.int32),
          pltpu.VMEM((C,), jnp.int32),
          pltpu.VMEM((C,), jnp.float32),
          pltpu.VMEM((C,), jnp.float32),
          pltpu.VMEM((C,), jnp.float32),
      ],
      compiler_params=pltpu.CompilerParams(needs_layout_passes=False),
  )
  return f(px, py, pz, dst_pad, src_pad)


def _sc_gather_body(nchunks, tdst_ref, tsrc_ref, dst_ref, src_ref, edst_ref,
                    esrc_ref, idxd, idxs, bufd, bufs, wsem):
  c = lax.axis_index("c")
  s = lax.axis_index("s")
  w = s * 2 + c

  def body(i, carry):
    b = i % 2
    off = (i * NW + w) * C

    @pl.when(i >= 2)
    def _():
      pltpu.make_async_copy(bufd.at[b], edst_ref.at[pl.ds(0, C)],
                            wsem.at[b]).wait()
      pltpu.make_async_copy(bufs.at[b], esrc_ref.at[pl.ds(0, C)],
                            wsem.at[b]).wait()

    pltpu.sync_copy(dst_ref.at[pl.ds(off, C)], idxd)
    pltpu.sync_copy(src_ref.at[pl.ds(off, C)], idxs)
    pltpu.sync_copy(tdst_ref.at[idxd], bufd.at[b])
    pltpu.sync_copy(tsrc_ref.at[idxs], bufs.at[b])
    pltpu.make_async_copy(bufd.at[b], edst_ref.at[pl.ds(off, C)],
                          wsem.at[b]).start()
    pltpu.make_async_copy(bufs.at[b], esrc_ref.at[pl.ds(off, C)],
                          wsem.at[b]).start()
    return carry

  lax.fori_loop(0, nchunks, body, 0)
  for b in (0, 1):
    pltpu.make_async_copy(bufd.at[b], edst_ref.at[pl.ds(0, C)],
                          wsem.at[b]).wait()
    pltpu.make_async_copy(bufs.at[b], esrc_ref.at[pl.ds(0, C)],
                          wsem.at[b]).wait()


def _sc_gather(tdst, tsrc, dst_pad, src_pad):
  eh = dst_pad.shape[0]
  nchunks = eh // (C * NW)
  mesh = plsc.VectorSubcoreMesh(core_axis_name="c", subcore_axis_name="s")
  f = pl.kernel(
      functools.partial(_sc_gather_body, nchunks),
      out_type=[
          jax.ShapeDtypeStruct((eh, TDT), jnp.float32),
          jax.ShapeDtypeStruct((eh, TDT), jnp.int32),
      ],
      mesh=mesh,
      scratch_types=[
          pltpu.VMEM((C,), jnp.int32),
          pltpu.VMEM((C,), jnp.int32),
          pltpu.VMEM((2, C, TDT), jnp.float32),
          pltpu.VMEM((2, C, TDT), jnp.int32),
          pltpu.SemaphoreType.DMA((2,)),
      ],
  )
  return f(tdst, tsrc, dst_pad, src_pad)


def _edge_body(nreal, ed_ref, es_ref, rel_ref, r1_ref, r2_ref, rv_ref,
               wsh_ref, out_ref):
  i = pl.program_id(0)

  @pl.when(i >= nreal)
  def _():
    out_ref[...] = jnp.zeros_like(out_ref)

  @pl.when(i < nreal)
  def _():
    q = ed_ref[...]
    wu = lax.bitcast_convert_type(es_ref[...], jnp.uint32)
    k0 = lax.bitcast_convert_type(
        (wu & 0xFFFF).astype(jnp.uint16), jnp.bfloat16).astype(jnp.float32)
    v0 = lax.bitcast_convert_type(
        (wu >> 16).astype(jnp.uint16), jnp.bfloat16).astype(jnp.float32)

    relT = rel_ref[...]
    rx = relT[0:1, :]
    ry = relT[1:2, :]
    rz = relT[2:3, :]
    r2T = rx * rx + ry * ry + rz * rz + 1e-12
    rT = jnp.sqrt(r2T)
    inv_r = 1.0 / (rT + 1e-9)
    dx = rx * inv_r
    dy = ry * inv_r
    dz = rz * inv_r

    centers = lax.broadcasted_iota(jnp.int32, (NB, BE), 0).astype(
        jnp.float32) * (RMAX / (NB - 1))
    width = RMAX / NB
    tT = (jnp.broadcast_to(rT, (NB, BE)) - centers) * (1.0 / width)
    rbfT = jnp.exp(-(tT * tT))
    envT = jnp.where(rT < RMAX, 0.5 * (jnp.cos(PI / RMAX * rT) + 1.0), 0.0)
    rbT = rbfT * envT
    rb = jnp.transpose(rbT)

    hidden = _silu(jnp.dot(rb, r1_ref[...], precision=_HI,
                           preferred_element_type=jnp.float32))
    rk = jnp.dot(hidden, r2_ref[...], precision=_HI,
                 preferred_element_type=jnp.float32)
    rvv = jnp.dot(hidden, rv_ref[...], precision=_HI,
                  preferred_element_type=jnp.float32)

    one = jnp.ones_like(dx)
    shT = jnp.concatenate([
        one, dx, dy, dz,
        1.7320508 * dx * dy, 1.7320508 * dy * dz,
        0.5 * (3.0 * dz * dz - 1.0),
        1.7320508 * dx * dz, 0.8660254 * (dx * dx - dy * dy),
        jnp.zeros((7, BE), jnp.float32),
    ], axis=0)
    sh = jnp.transpose(shT)
    shw = jnp.dot(sh, wsh_ref[...], precision=_HI,
                  preferred_element_type=jnp.float32)

    k = k0 * rk + shw
    v = v0 * rvv
    logits = jnp.sum(q * k, axis=1, keepdims=True) * (DH ** -0.5)
    ex = jnp.exp(logits)

    zpad = jnp.zeros((BE, 63), jnp.float32)
    out_ref[0] = jnp.concatenate([ex * v[:, :64], ex, zpad], axis=1)
    out_ref[1] = jnp.concatenate([ex * v[:, 64:], ex, zpad], axis=1)


def _edge(edst, esrc, rel8, r1, r2, rv, wsh16, nreal):
  eh = edst.shape[0]
  grid = eh // BE
  return pl.pallas_call(
      functools.partial(_edge_body, nreal),
      grid=(grid,),
      in_specs=[
          pl.BlockSpec((BE, TDT), lambda i: (i, 0)),
          pl.BlockSpec((BE, TDT), lambda i: (i, 0)),
          pl.BlockSpec((8, BE), lambda i: (0, i)),
          pl.BlockSpec((NB, 64), lambda i: (0, 0)),
          pl.BlockSpec((64, DH), lambda i: (0, 0)),
          pl.BlockSpec((64, DH), lambda i: (0, 0)),
          pl.BlockSpec((16, DH), lambda i: (0, 0)),
      ],
      out_specs=pl.BlockSpec((2, BE, TD), lambda i: (0, i, 0)),
      out_shape=jax.ShapeDtypeStruct((2, eh, TD), jnp.float32),
  )(edst, esrc, rel8, r1, r2, rv, wsh16)


def _sc_scatter_body(nchunks, ev_ref, dst_ref, zrows_ref, out_ref, idxb, buf,
                     acc, asem):
  c = lax.axis_index("c")
  s = lax.axis_index("s")
  rpt = N_PAD // 16
  base = s * rpt
  pltpu.sync_copy(zrows_ref.at[pl.ds(0, rpt)], acc.at[pl.ds(base, rpt)])
  plsc.subcore_barrier()

  def body(i, carry):
    b = i % 2
    off = (i * 16 + s) * C

    @pl.when(i >= 2)
    def _():
      pltpu.make_async_copy(buf.at[b], acc.at[idxb.at[b]],
                            asem.at[b]).wait()

    pltpu.sync_copy(dst_ref.at[pl.ds(off, C)], idxb.at[b])
    pltpu.sync_copy(ev_ref.at[c, pl.ds(off, C)], buf.at[b])
    pltpu.make_async_copy(buf.at[b], acc.at[idxb.at[b]],
                          asem.at[b]).start(add=True)
    return carry

  lax.fori_loop(0, nchunks, body, 0)
  for b in (0, 1):
    pltpu.make_async_copy(buf.at[b], acc.at[idxb.at[b]], asem.at[b]).wait()
  plsc.subcore_barrier()
  pltpu.sync_copy(acc.at[pl.ds(base, rpt)], out_ref.at[c, pl.ds(base, rpt)])


def _sc_scatter(ev, dst_pad, zrows):
  nchunks = dst_pad.shape[0] // (C * 16)
  mesh = plsc.VectorSubcoreMesh(core_axis_name="c", subcore_axis_name="s")
  f = pl.kernel(
      functools.partial(_sc_scatter_body, nchunks),
      out_type=jax.ShapeDtypeStruct((2, N_PAD, TD), jnp.float32),
      mesh=mesh,
      scratch_types=[
          pltpu.VMEM((2, C), jnp.int32),
          pltpu.VMEM((2, C, TD), jnp.float32),
          pltpu.VMEM_SHARED((N_PAD, TD), jnp.float32),
          pltpu.SemaphoreType.DMA((2,)),
      ],
  )
  return f(ev, dst_pad, zrows)


def _pool_body(p_ref, batch_ref, out_ref, acc):
  i = pl.program_id(0)

  @pl.when(i == 0)
  def _():
    acc[...] = jnp.zeros_like(acc)

  p0 = p_ref[0]
  p1 = p_ref[1]
  den = p0[:, 64:65]
  h = jnp.concatenate([p0[:, :64], p1[:, :64]], axis=1) * (1.0 / (den + 1e-9))

  bt = batch_ref[0]
  oh = (lax.broadcasted_iota(jnp.int32, (G, BN), 0) == bt).astype(jnp.float32)
  hext = jnp.concatenate(
      [h, jnp.ones((BN, 1), jnp.float32), jnp.zeros((BN, 127), jnp.float32)],
      axis=1)
  acc[...] += jnp.dot(oh, hext, precision=_HI,
                      preferred_element_type=jnp.float32)

  @pl.when(i == (N_PAD // BN) - 1)
  def _():
    cnt = acc[:, 128:129]
    out_ref[...] = acc[:, :128] * (1.0 / jnp.maximum(cnt, 1.0))


def _pool(partials, batch3):
  grid = N_PAD // BN
  return pl.pallas_call(
      _pool_body,
      grid=(grid,),
      in_specs=[
          pl.BlockSpec((2, BN, TD), lambda i: (0, i, 0)),
          pl.BlockSpec((1, 1, BN), lambda i: (i, 0, 0)),
      ],
      out_specs=pl.BlockSpec((G, DH), lambda i: (0, 0)),
      out_shape=jax.ShapeDtypeStruct((G, DH), jnp.float32),
      scratch_shapes=[pltpu.VMEM((G, 256), jnp.float32)],
      compiler_params=pltpu.CompilerParams(
          dimension_semantics=("arbitrary",)),
  )(partials, batch3)


def kernel(x, pos, edge_index, batch, W_emb, b_emb,
           Wq0, Wk0, Wv0, R1_0, R2_0, Rv_0, Wsh0,
           Wq1, Wk1, Wv1, R1_1, R2_1, Rv_1, Wsh1):
  src = edge_index[0].astype(jnp.int32)
  dst = edge_index[1].astype(jnp.int32)
  src_pad = jnp.pad(src, (0, E_PAD - E))
  dst_pad = jnp.pad(dst, (0, E_PAD - E))

  x_pad = jnp.pad(x, ((0, N_PAD - N), (0, 0)))
  pos_pad = jnp.pad(pos, ((0, N_PAD - N), (0, 0)))
  px = pos_pad[:, 0]
  py = pos_pad[:, 1]
  pz = pos_pad[:, 2]
  batch3 = jnp.pad(batch.astype(jnp.int32), (0, N_PAD - N),
                   constant_values=G).reshape(N_PAD // BN, 1, BN)
  bemb2 = b_emb.reshape(1, 64)
  wsh0_16 = jnp.pad(Wsh0, ((0, 7), (0, 0)))
  wsh1_16 = jnp.pad(Wsh1, ((0, 7), (0, 0)))
  zrows = jnp.zeros((N_PAD // 16, TD), jnp.float32)

  rel8 = _sc_geo(px, py, pz, dst_pad, src_pad)

  H1 = 40 * C * NW
  dst_a, dst_b = dst_pad[:H1], dst_pad[H1:]
  src_a, src_b = src_pad[:H1], src_pad[H1:]
  rel_a = lax.slice(rel8, (0, 0), (8, H1))
  rel_b = lax.slice(rel8, (0, H1), (8, E_PAD))
  nreal_a = H1 // BE
  nreal_b = (E - H1) // BE

  def layer(tdst, tsrc, r1, r2, rv, wsh16):
    eda, esa = _sc_gather(tdst, tsrc, dst_a, src_a)
    edb, esb = _sc_gather(tdst, tsrc, dst_b, src_b)
    ev_a = _edge(eda, esa, rel_a, r1, r2, rv, wsh16, nreal_a)
    ev_b = _edge(edb, esb, rel_b, r1, r2, rv, wsh16, nreal_b)
    pa = _sc_scatter(ev_a, dst_a, zrows)
    pb = _sc_scatter(ev_b, dst_b, zrows)
    return pa + pb

  tdst, tsrc = _node0(x_pad, W_emb, bemb2, Wq0, Wk0, Wv0)
  part0 = layer(tdst, tsrc, R1_0, R2_0, Rv_0, wsh0_16)

  tdst, tsrc = _node1(part0, Wq1, Wk1, Wv1)
  part1 = layer(tdst, tsrc, R1_1, R2_1, Rv_1, wsh1_16)

  return _pool(part1, batch3)
